# baseline (speedup 1.0000x reference)
import functools

import jax
import jax.numpy as jnp
from jax import lax
from jax.experimental import pallas as pl
from jax.experimental.pallas import tpu as pltpu
from jax.experimental.pallas import tpu_sc as plsc

N = 10000
E = 320000
D_IN = 128
D_HID = 64

NC = 2
NS = 16
NW = NC * NS
CHUNK = 128
NLOC = 80
NCHUNKS_PAD = NW * NLOC
E_PAD = NCHUNKS_PAD * CHUNK
PAD_BIN = 16
N_ACC = N + PAD_BIN
ROWS_PER_SUB = 624
TAIL_ROWS = N - NS * ROWS_PER_SUB
DEG_W = 16

G_MSG = 4
NG_MSG = NLOC // G_MSG
G_PAIR = 2
NG_PAIR = NLOC // G_PAIR


def _worker_id():
    return lax.axis_index("s") * NC + lax.axis_index("c")


def _copy_share(src, dst, s):
    r0 = s * ROWS_PER_SUB
    pltpu.sync_copy(src.at[pl.ds(r0, ROWS_PER_SUB)],
                    dst.at[pl.ds(r0, ROWS_PER_SUB)])

    @pl.when(s == NS - 1)
    def _():
        t0 = NS * ROWS_PER_SUB
        pltpu.sync_copy(src.at[pl.ds(t0, TAIL_ROWS)],
                        dst.at[pl.ds(t0, TAIL_ROWS)])


def _writeback(acc, out0, out1, c, s):
    @pl.when(c == 0)
    def _():
        _copy_share(acc, out0, s)

    @pl.when(c == 1)
    def _():
        _copy_share(acc, out1, s)


@functools.lru_cache(maxsize=None)
def _sc_mesh():
    return plsc.VectorSubcoreMesh(
        core_axis_name="c", subcore_axis_name="s", num_cores=NC, num_subcores=NS
    )


@functools.lru_cache(maxsize=None)
def _deg_sc():
    @functools.partial(
        pl.kernel,
        out_type=(
            jax.ShapeDtypeStruct((N, DEG_W), jnp.float32),
            jax.ShapeDtypeStruct((N, DEG_W), jnp.float32),
        ),
        mesh=_sc_mesh(),
        scratch_types=[
            pltpu.VMEM((NLOC, CHUNK), jnp.int32),
            pltpu.VMEM((CHUNK, DEG_W), jnp.float32),
            pltpu.VMEM_SHARED((N_ACC, DEG_W), jnp.float32),
            pltpu.SemaphoreType.DMA,
        ],
        compiler_params=pltpu.CompilerParams(use_tc_tiling_on_sc=False),
    )
    def deg_kernel(dst2d_hbm, ones_hbm, zeros_hbm, out0, out1,
                   didx, ones_v, acc, sem):
        c = lax.axis_index("c")
        s = lax.axis_index("s")
        w = _worker_id()
        c0 = pl.multiple_of(w * NLOC, 8)
        pltpu.sync_copy(dst2d_hbm.at[pl.ds(c0, NLOC)], didx)
        pltpu.sync_copy(ones_hbm, ones_v)
        _copy_share(zeros_hbm, acc, s)
        plsc.subcore_barrier()

        def fire16(t, carry):
            for j in range(16):
                li = t * 16 + j
                pltpu.make_async_copy(ones_v, acc.at[didx.at[li]], sem).start(add=True)
            return carry

        def drain16(t, carry):
            for j in range(16):
                li = t * 16 + j
                pltpu.make_async_copy(ones_v, acc.at[didx.at[li]], sem).wait()
            return carry

        lax.fori_loop(0, NLOC // 16, fire16, 0)
        lax.fori_loop(0, NLOC // 16, drain16, 0)
        plsc.subcore_barrier()
        _writeback(acc, out0, out1, c, s)

    return deg_kernel


@functools.lru_cache(maxsize=None)
def _msg_sc():
    @functools.partial(
        pl.kernel,
        out_type=(
            jax.ShapeDtypeStruct((N, D_HID), jnp.float32),
            jax.ShapeDtypeStruct((N, D_HID), jnp.float32),
        ),
        mesh=_sc_mesh(),
        scratch_types=[
            pltpu.VMEM((NLOC * CHUNK,), jnp.int32),
            pltpu.VMEM((NLOC, CHUNK), jnp.int32),
            pltpu.VMEM((2 * G_MSG, CHUNK, D_HID), jnp.float32),
            pltpu.VMEM_SHARED((N_ACC, D_HID), jnp.float32),
            pltpu.SemaphoreType.DMA,
            pltpu.SemaphoreType.DMA,
            pltpu.SemaphoreType.DMA,
            pltpu.SemaphoreType.DMA,
        ],
        compiler_params=pltpu.CompilerParams(use_tc_tiling_on_sc=False),
    )
    def msg_kernel(y_hbm, src1d_hbm, dst2d_hbm, zeros_hbm, out0, out1,
                   sidx, didx, rows, acc, sga, sgb, ssa, ssb):
        c = lax.axis_index("c")
        s = lax.axis_index("s")
        w = _worker_id()
        c0 = pl.multiple_of(w * NLOC, 8)
        pltpu.sync_copy(src1d_hbm.at[pl.ds(c0 * CHUNK, NLOC * CHUNK)], sidx)
        pltpu.sync_copy(dst2d_hbm.at[pl.ds(c0, NLOC)], didx)
        _copy_share(zeros_hbm, acc, s)

        def gather_desc(g, bank, j, sem):
            li = g * G_MSG + j
            return pltpu.make_async_copy(
                y_hbm.at[sidx.at[pl.ds(li * CHUNK, CHUNK)]],
                rows.at[bank * G_MSG + j], sem)

        def scatter_desc(g, bank, j, sem):
            li = g * G_MSG + j
            return pltpu.make_async_copy(
                rows.at[bank * G_MSG + j], acc.at[didx.at[li]], sem)

        def fire_gathers(g, bank, sem):
            for j in range(G_MSG):
                gather_desc(g, bank, j, sem).start()

        def drain_gathers(g, bank, sem):
            for j in range(G_MSG):
                gather_desc(g, bank, j, sem).wait()

        def fire_scatters(g, bank, sem):
            for j in range(G_MSG):
                scatter_desc(g, bank, j, sem).start(add=True)

        def drain_scatters(g, bank, sem):
            for j in range(G_MSG):
                scatter_desc(g, bank, j, sem).wait()

        fire_gathers(0, 0, sga)
        fire_gathers(1, 1, sgb)
        plsc.subcore_barrier()

        def body(t, carry):
            g0 = 2 * t
            g1 = g0 + 1
            drain_gathers(g0, 0, sga)
            fire_scatters(g0, 0, ssa)
            drain_gathers(g1, 1, sgb)
            fire_scatters(g1, 1, ssb)
            drain_scatters(g0, 0, ssa)
            fire_gathers(g0 + 2, 0, sga)
            drain_scatters(g1, 1, ssb)
            fire_gathers(g1 + 2, 1, sgb)
            return carry

        lax.fori_loop(0, NG_MSG // 2 - 1, body, 0)
        g0 = NG_MSG - 2
        g1 = NG_MSG - 1
        drain_gathers(g0, 0, sga)
        fire_scatters(g0, 0, ssa)
        drain_gathers(g1, 1, sgb)
        fire_scatters(g1, 1, ssb)
        drain_scatters(g0, 0, ssa)
        drain_scatters(g1, 1, ssb)
        plsc.subcore_barrier()
        _writeback(acc, out0, out1, c, s)

    return msg_kernel


_NSLOT = 3
_GRP = CHUNK // 16
_TSTRIDE = 65


@functools.lru_cache(maxsize=None)
def _edge_head_sc():
    @functools.partial(
        pl.kernel,
        out_type=jax.ShapeDtypeStruct((2, E_PAD), jnp.float32),
        mesh=_sc_mesh(),
        scratch_types=[
            pltpu.VMEM((NLOC * CHUNK,), jnp.int32),
            pltpu.VMEM((NLOC * CHUNK,), jnp.int32),
            pltpu.VMEM((2, D_HID, 16), jnp.float32),
            pltpu.VMEM((2, 16), jnp.float32),
            pltpu.VMEM((_NSLOT, CHUNK, D_HID), jnp.float32),
            pltpu.VMEM((_NSLOT, CHUNK, D_HID), jnp.float32),
            pltpu.VMEM((_NSLOT, CHUNK * _TSTRIDE), jnp.float32),
            pltpu.VMEM((_NSLOT, 2, CHUNK), jnp.float32),
            pltpu.SemaphoreType.DMA((_NSLOT,)),
            pltpu.SemaphoreType.DMA((_NSLOT,)),
        ],
        compiler_params=pltpu.CompilerParams(use_tc_tiling_on_sc=False,
                                             needs_layout_passes=False),
    )
    def edge_head_kernel(ha_hbm, hb_hbm, src1d_hbm, dst1d_hbm, wl2_hbm, bl2_hbm,
                         z_out, sidx, didx, wl2_v, bl2_v, rowsa, rowsb, tbuf,
                         zbuf, sg, sw):
        w = _worker_id()
        c0 = pl.multiple_of(w * NLOC, 8)
        pltpu.sync_copy(src1d_hbm.at[pl.ds(c0 * CHUNK, NLOC * CHUNK)], sidx)
        pltpu.sync_copy(dst1d_hbm.at[pl.ds(c0 * CHUNK, NLOC * CHUNK)], didx)
        pltpu.sync_copy(wl2_hbm, wl2_v)
        pltpu.sync_copy(bl2_hbm, bl2_v)

        iota16 = lax.iota(jnp.int32, 16)
        col_idx = [iota16 * _TSTRIDE + (g * 16 * _TSTRIDE) for g in range(_GRP)]

        def gather_descs(li, k):
            return (
                pltpu.make_async_copy(
                    ha_hbm.at[sidx.at[pl.ds(li * CHUNK, CHUNK)]],
                    rowsa.at[k], sg.at[k]),
                pltpu.make_async_copy(
                    hb_hbm.at[didx.at[pl.ds(li * CHUNK, CHUNK)]],
                    rowsb.at[k], sg.at[k]),
            )

        def zwrite_desc(li, k):
            base = (c0 + li) * CHUNK
            return pltpu.make_async_copy(
                zbuf.at[k], z_out.at[:, pl.ds(base, CHUNK)], sw.at[k])

        for dsc in gather_descs(0, 0) + gather_descs(1, 1):
            dsc.start()

        def body(li, carry):
            k = lax.rem(li, _NSLOT)
            for dsc in gather_descs(li, k):
                dsc.wait()

            @pl.when(li + 2 < NLOC)
            def _():
                k2 = lax.rem(li + 2, _NSLOT)
                for dsc in gather_descs(li + 2, k2):
                    dsc.start()

            @pl.when(li >= _NSLOT)
            def _():
                zwrite_desc(li, k).wait()

            def ebody(e8, carry2):
                for u in range(8):
                    e = e8 * 8 + u
                    for q in range(D_HID // 16):
                        av = rowsa[k, e, pl.ds(q * 16, 16)]
                        bv = rowsb[k, e, pl.ds(q * 16, 16)]
                        tbuf[k, pl.ds(e * _TSTRIDE + q * 16, 16)] = jnp.maximum(
                            av + bv, 0.0)
                return carry2

            lax.fori_loop(0, CHUNK // 8, ebody, 0)

            ksplat = jnp.zeros((16,), jnp.int32) + k

            def dbody(d, accs):
                w0d = wl2_v[0, d]
                w1d = wl2_v[1, d]
                out = []
                for g in range(_GRP):
                    r = plsc.load_gather(tbuf, [ksplat, col_idx[g] + d])
                    out.append(accs[2 * g] + r * w0d)
                    out.append(accs[2 * g + 1] + r * w1d)
                return tuple(out)

            init = tuple(bl2_v[cc] for _ in range(_GRP) for cc in (0, 1))
            accs = lax.fori_loop(0, D_HID, dbody, init)
            for g in range(_GRP):
                zbuf[k, 0, pl.ds(g * 16, 16)] = accs[2 * g]
                zbuf[k, 1, pl.ds(g * 16, 16)] = accs[2 * g + 1]

            zwrite_desc(li, k).start()
            return carry

        lax.fori_loop(0, NLOC, body, 0)
        for li in range(NLOC - _NSLOT, NLOC):
            zwrite_desc(li, li % _NSLOT).wait()

    return edge_head_kernel


_MBLK = 2000
_EBLK = 4000
_LSMBLK = 6400


def _dis_block(d0_ref, d1_ref):
    deg = d0_ref[:, 0:1] + d1_ref[:, 0:1] + 1.0
    return lax.rsqrt(deg)


def _mm1_body(x_ref, w_ref, d0_ref, d1_ref, o_ref):
    dis = _dis_block(d0_ref, d1_ref)
    xw = jnp.dot(x_ref[...], w_ref[...], preferred_element_type=jnp.float32)
    o_ref[...] = xw * dis


def _combine1_body(s0_ref, s1_ref, y_ref, d0_ref, d1_ref, b_ref, w_ref, o_ref):
    dis = _dis_block(d0_ref, d1_ref)
    h = jnp.maximum((s0_ref[...] + s1_ref[...] + y_ref[...]) * dis + b_ref[...], 0.0)
    o_ref[...] = jnp.dot(h, w_ref[...], preferred_element_type=jnp.float32) * dis


def _combine2_body(s0_ref, s1_ref, y_ref, d0_ref, d1_ref, b_ref, wl1_ref,
                   bl1_ref, oa_ref, ob_ref):
    dis = _dis_block(d0_ref, d1_ref)
    h = jnp.maximum((s0_ref[...] + s1_ref[...] + y_ref[...]) * dis + b_ref[...], 0.0)
    oa_ref[...] = (jnp.dot(h, wl1_ref[0:D_HID, :], preferred_element_type=jnp.float32)
                   + bl1_ref[...])
    ob_ref[...] = jnp.dot(h, wl1_ref[D_HID:2 * D_HID, :],
                          preferred_element_type=jnp.float32)


def _lsm_body(z_ref, o_ref):
    z0 = z_ref[0:1, :]
    z1 = z_ref[1:2, :]
    m = jnp.maximum(z0, z1)
    lse = m + jnp.log(jnp.exp(z0 - m) + jnp.exp(z1 - m))
    o_ref[...] = jnp.concatenate([z0 - lse, z1 - lse], axis=0)


def _node_spec(width):
    return pl.BlockSpec((_MBLK, width), lambda i: (i, 0))


def _full_spec(shape):
    return pl.BlockSpec(shape, lambda i: tuple(0 for _ in shape))


def kernel(x, edge_index, W1, b1, W2, b2, Wl1, bl1, Wl2, bl2):
    src = edge_index[0].astype(jnp.int32)
    dst = edge_index[1].astype(jnp.int32)

    npad = E_PAD - E
    pad_src = (jnp.arange(npad, dtype=jnp.int32) * 37) % N
    pad_dst = N + (jnp.arange(npad, dtype=jnp.int32) % PAD_BIN)
    srcp = jnp.concatenate([src, pad_src])
    dstp = jnp.concatenate([dst, pad_dst])
    dstp2d = dstp.reshape(NCHUNKS_PAD, CHUNK)

    zeros_deg = jnp.zeros((N, DEG_W), jnp.float32)
    ones_deg = jnp.ones((CHUNK, DEG_W), jnp.float32)
    zeros_hid = jnp.zeros((N, D_HID), jnp.float32)
    b1r = b1.reshape(1, D_HID)
    b2r = b2.reshape(1, D_HID)
    bl1r = bl1.reshape(1, D_HID)
    bl2r = bl2.reshape(1, 2)

    deg0, deg1 = _deg_sc()(dstp2d, ones_deg, zeros_deg)

    y1 = pl.pallas_call(
        _mm1_body,
        grid=(N // _MBLK,),
        in_specs=[
            _node_spec(D_IN),
            _full_spec((D_IN, D_HID)),
            _node_spec(DEG_W),
            _node_spec(DEG_W),
        ],
        out_specs=_node_spec(D_HID),
        out_shape=jax.ShapeDtypeStruct((N, D_HID), jnp.float32),
    )(x, W1, deg0, deg1)

    s1a, s1b = _msg_sc()(y1, srcp, dstp2d, zeros_hid)

    y2 = pl.pallas_call(
        _combine1_body,
        grid=(N // _MBLK,),
        in_specs=[
            _node_spec(D_HID),
            _node_spec(D_HID),
            _node_spec(D_HID),
            _node_spec(DEG_W),
            _node_spec(DEG_W),
            _full_spec((1, D_HID)),
            _full_spec((D_HID, D_HID)),
        ],
        out_specs=_node_spec(D_HID),
        out_shape=jax.ShapeDtypeStruct((N, D_HID), jnp.float32),
    )(s1a, s1b, y1, deg0, deg1, b1r, W2)

    s2a, s2b = _msg_sc()(y2, srcp, dstp2d, zeros_hid)

    ha, hb = pl.pallas_call(
        _combine2_body,
        grid=(N // _MBLK,),
        in_specs=[
            _node_spec(D_HID),
            _node_spec(D_HID),
            _node_spec(D_HID),
            _node_spec(DEG_W),
            _node_spec(DEG_W),
            _full_spec((1, D_HID)),
            _full_spec((D_IN, D_HID)),
            _full_spec((1, D_HID)),
        ],
        out_specs=(_node_spec(D_HID), _node_spec(D_HID)),
        out_shape=(
            jax.ShapeDtypeStruct((N, D_HID), jnp.float32),
            jax.ShapeDtypeStruct((N, D_HID), jnp.float32),
        ),
    )(s2a, s2b, y2, deg0, deg1, b2r, Wl1, bl1r)

    wl2b = jnp.broadcast_to(Wl2.T[:, :, None], (2, D_HID, 16)).astype(jnp.float32)
    bl2b = jnp.broadcast_to(bl2[:, None], (2, 16)).astype(jnp.float32)
    z = _edge_head_sc()(ha, hb, srcp, dstp, wl2b, bl2b)

    out2 = pl.pallas_call(
        _lsm_body,
        grid=(E // _LSMBLK,),
        in_specs=[pl.BlockSpec((2, _LSMBLK), lambda i: (0, i))],
        out_specs=pl.BlockSpec((2, _LSMBLK), lambda i: (0, i)),
        out_shape=jax.ShapeDtypeStruct((2, E), jnp.float32),
    )(z)

    return out2.T


# speedup vs baseline: 1.0428x; 1.0428x over previous
import functools

import jax
import jax.numpy as jnp
from jax import lax
from jax.experimental import pallas as pl
from jax.experimental.pallas import tpu as pltpu
from jax.experimental.pallas import tpu_sc as plsc

N = 10000
E = 320000
D_IN = 128
D_HID = 64

NC = 2
NS = 16
NW = NC * NS
CHUNK = 128
NLOC = 80
NCHUNKS_PAD = NW * NLOC
E_PAD = NCHUNKS_PAD * CHUNK
PAD_BIN = 16
N_ACC = N + PAD_BIN
ROWS_PER_SUB = 624
TAIL_ROWS = N - NS * ROWS_PER_SUB
DEG_W = 16

G_MSG = 4
NG_MSG = NLOC // G_MSG
G_PAIR = 2
NG_PAIR = NLOC // G_PAIR


def _worker_id():
    return lax.axis_index("s") * NC + lax.axis_index("c")


def _copy_share(src, dst, s):
    r0 = s * ROWS_PER_SUB
    pltpu.sync_copy(src.at[pl.ds(r0, ROWS_PER_SUB)],
                    dst.at[pl.ds(r0, ROWS_PER_SUB)])

    @pl.when(s == NS - 1)
    def _():
        t0 = NS * ROWS_PER_SUB
        pltpu.sync_copy(src.at[pl.ds(t0, TAIL_ROWS)],
                        dst.at[pl.ds(t0, TAIL_ROWS)])


def _writeback(acc, out0, out1, c, s):
    @pl.when(c == 0)
    def _():
        _copy_share(acc, out0, s)

    @pl.when(c == 1)
    def _():
        _copy_share(acc, out1, s)


@functools.lru_cache(maxsize=None)
def _sc_mesh():
    return plsc.VectorSubcoreMesh(
        core_axis_name="c", subcore_axis_name="s", num_cores=NC, num_subcores=NS
    )


@functools.lru_cache(maxsize=None)
def _deg_sc():
    @functools.partial(
        pl.kernel,
        out_type=(
            jax.ShapeDtypeStruct((N, DEG_W), jnp.float32),
            jax.ShapeDtypeStruct((N, DEG_W), jnp.float32),
        ),
        mesh=_sc_mesh(),
        scratch_types=[
            pltpu.VMEM((NLOC, CHUNK), jnp.int32),
            pltpu.VMEM((CHUNK, DEG_W), jnp.float32),
            pltpu.VMEM_SHARED((N_ACC, DEG_W), jnp.float32),
            pltpu.SemaphoreType.DMA,
        ],
        compiler_params=pltpu.CompilerParams(use_tc_tiling_on_sc=False),
    )
    def deg_kernel(dst2d_hbm, ones_hbm, zeros_hbm, out0, out1,
                   didx, ones_v, acc, sem):
        c = lax.axis_index("c")
        s = lax.axis_index("s")
        w = _worker_id()
        c0 = pl.multiple_of(w * NLOC, 8)
        pltpu.sync_copy(dst2d_hbm.at[pl.ds(c0, NLOC)], didx)
        pltpu.sync_copy(ones_hbm, ones_v)
        _copy_share(zeros_hbm, acc, s)
        plsc.subcore_barrier()

        def fire16(t, carry):
            for j in range(16):
                li = t * 16 + j
                pltpu.make_async_copy(ones_v, acc.at[didx.at[li]], sem).start(add=True)
            return carry

        def drain16(t, carry):
            for j in range(16):
                li = t * 16 + j
                pltpu.make_async_copy(ones_v, acc.at[didx.at[li]], sem).wait()
            return carry

        lax.fori_loop(0, NLOC // 16, fire16, 0)
        lax.fori_loop(0, NLOC // 16, drain16, 0)
        plsc.subcore_barrier()
        _writeback(acc, out0, out1, c, s)

    return deg_kernel


@functools.lru_cache(maxsize=None)
def _msg_sc():
    @functools.partial(
        pl.kernel,
        out_type=(
            jax.ShapeDtypeStruct((N, D_HID), jnp.float32),
            jax.ShapeDtypeStruct((N, D_HID), jnp.float32),
        ),
        mesh=_sc_mesh(),
        scratch_types=[
            pltpu.VMEM((NLOC * CHUNK,), jnp.int32),
            pltpu.VMEM((NLOC, CHUNK), jnp.int32),
            pltpu.VMEM((2 * G_MSG, CHUNK, D_HID), jnp.float32),
            pltpu.VMEM_SHARED((N_ACC, D_HID), jnp.float32),
            pltpu.SemaphoreType.DMA,
            pltpu.SemaphoreType.DMA,
            pltpu.SemaphoreType.DMA,
            pltpu.SemaphoreType.DMA,
        ],
        compiler_params=pltpu.CompilerParams(use_tc_tiling_on_sc=False),
    )
    def msg_kernel(y_hbm, src1d_hbm, dst2d_hbm, zeros_hbm, out0, out1,
                   sidx, didx, rows, acc, sga, sgb, ssa, ssb):
        c = lax.axis_index("c")
        s = lax.axis_index("s")
        w = _worker_id()
        c0 = pl.multiple_of(w * NLOC, 8)
        pltpu.sync_copy(src1d_hbm.at[pl.ds(c0 * CHUNK, NLOC * CHUNK)], sidx)
        pltpu.sync_copy(dst2d_hbm.at[pl.ds(c0, NLOC)], didx)
        _copy_share(zeros_hbm, acc, s)

        def gather_desc(g, bank, j, sem):
            li = g * G_MSG + j
            return pltpu.make_async_copy(
                y_hbm.at[sidx.at[pl.ds(li * CHUNK, CHUNK)]],
                rows.at[bank * G_MSG + j], sem)

        def scatter_desc(g, bank, j, sem):
            li = g * G_MSG + j
            return pltpu.make_async_copy(
                rows.at[bank * G_MSG + j], acc.at[didx.at[li]], sem)

        def fire_gathers(g, bank, sem):
            for j in range(G_MSG):
                gather_desc(g, bank, j, sem).start()

        def drain_gathers(g, bank, sem):
            for j in range(G_MSG):
                gather_desc(g, bank, j, sem).wait()

        def fire_scatters(g, bank, sem):
            for j in range(G_MSG):
                scatter_desc(g, bank, j, sem).start(add=True)

        def drain_scatters(g, bank, sem):
            for j in range(G_MSG):
                scatter_desc(g, bank, j, sem).wait()

        fire_gathers(0, 0, sga)
        fire_gathers(1, 1, sgb)
        plsc.subcore_barrier()

        def body(t, carry):
            g0 = 2 * t
            g1 = g0 + 1
            drain_gathers(g0, 0, sga)
            fire_scatters(g0, 0, ssa)
            drain_gathers(g1, 1, sgb)
            fire_scatters(g1, 1, ssb)
            drain_scatters(g0, 0, ssa)
            fire_gathers(g0 + 2, 0, sga)
            drain_scatters(g1, 1, ssb)
            fire_gathers(g1 + 2, 1, sgb)
            return carry

        lax.fori_loop(0, NG_MSG // 2 - 1, body, 0)
        g0 = NG_MSG - 2
        g1 = NG_MSG - 1
        drain_gathers(g0, 0, sga)
        fire_scatters(g0, 0, ssa)
        drain_gathers(g1, 1, sgb)
        fire_scatters(g1, 1, ssb)
        drain_scatters(g0, 0, ssa)
        drain_scatters(g1, 1, ssb)
        plsc.subcore_barrier()
        _writeback(acc, out0, out1, c, s)

    return msg_kernel


_NSLOT = 3
_GRP = CHUNK // 16
_TSTRIDE = 65


@functools.lru_cache(maxsize=None)
def _edge_head_sc():
    slot_scratch = []
    for _ in range(_NSLOT):
        slot_scratch += [
            pltpu.VMEM((CHUNK, D_HID), jnp.float32),
            pltpu.VMEM((CHUNK, D_HID), jnp.float32),
            pltpu.VMEM((CHUNK * _TSTRIDE,), jnp.float32),
            pltpu.VMEM((2, CHUNK), jnp.float32),
            pltpu.SemaphoreType.DMA,
            pltpu.SemaphoreType.DMA,
        ]

    @functools.partial(
        pl.kernel,
        out_type=jax.ShapeDtypeStruct((2, E_PAD), jnp.float32),
        mesh=_sc_mesh(),
        scratch_types=[
            pltpu.VMEM((NLOC * CHUNK,), jnp.int32),
            pltpu.VMEM((NLOC * CHUNK,), jnp.int32),
            pltpu.VMEM((2, D_HID, 16), jnp.float32),
            pltpu.VMEM((2, 16), jnp.float32),
        ] + slot_scratch,
        compiler_params=pltpu.CompilerParams(use_tc_tiling_on_sc=False,
                                             needs_layout_passes=False),
    )
    def edge_head_kernel(ha_hbm, hb_hbm, src1d_hbm, dst1d_hbm, wl2_hbm, bl2_hbm,
                         z_out, sidx, didx, wl2_v, bl2_v, *slot_refs):
        slots = [tuple(slot_refs[i * 6:(i + 1) * 6]) for i in range(_NSLOT)]
        w = _worker_id()
        c0 = pl.multiple_of(w * NLOC, 8)
        pltpu.sync_copy(src1d_hbm.at[pl.ds(c0 * CHUNK, NLOC * CHUNK)], sidx)
        pltpu.sync_copy(dst1d_hbm.at[pl.ds(c0 * CHUNK, NLOC * CHUNK)], didx)
        pltpu.sync_copy(wl2_hbm, wl2_v)
        pltpu.sync_copy(bl2_hbm, bl2_v)

        iota16 = lax.iota(jnp.int32, 16)
        col_idx = [iota16 * _TSTRIDE + (g * 16 * _TSTRIDE) for g in range(_GRP)]

        def gather_descs(li, k):
            ra, rb, _, _, sg, _ = slots[k]
            return (
                pltpu.make_async_copy(
                    ha_hbm.at[sidx.at[pl.ds(li * CHUNK, CHUNK)]], ra, sg),
                pltpu.make_async_copy(
                    hb_hbm.at[didx.at[pl.ds(li * CHUNK, CHUNK)]], rb, sg),
            )

        def zwrite_desc(li, k):
            _, _, _, zt, _, sw = slots[k]
            base = (c0 + li) * CHUNK
            return pltpu.make_async_copy(
                zt, z_out.at[:, pl.ds(base, CHUNK)], sw)

        def compute_chunk(k):
            ra, rb, tb, zt, _, _ = slots[k]

            def ebody(e8, carry):
                for u in range(8):
                    e = e8 * 8 + u
                    for q in range(D_HID // 16):
                        av = ra[e, pl.ds(q * 16, 16)]
                        bv = rb[e, pl.ds(q * 16, 16)]
                        tb[pl.ds(e * _TSTRIDE + q * 16, 16)] = jnp.maximum(
                            av + bv, 0.0)
                return carry

            lax.fori_loop(0, CHUNK // 8, ebody, 0)

            def dbody(d2, accs):
                out = list(accs)
                for dd in range(2):
                    d = d2 * 2 + dd
                    w0d = wl2_v[0, d]
                    w1d = wl2_v[1, d]
                    nxt = []
                    for g in range(_GRP):
                        r = plsc.load_gather(tb, [col_idx[g] + d])
                        nxt.append(out[2 * g] + r * w0d)
                        nxt.append(out[2 * g + 1] + r * w1d)
                    out = nxt
                return tuple(out)

            init = tuple(bl2_v[cc] for _ in range(_GRP) for cc in (0, 1))
            accs = lax.fori_loop(0, D_HID // 2, dbody, init)
            for g in range(_GRP):
                zt[0, pl.ds(g * 16, 16)] = accs[2 * g]
                zt[1, pl.ds(g * 16, 16)] = accs[2 * g + 1]

        def step(li, k, zdrain, prefetch):
            for dsc in gather_descs(li, k):
                dsc.wait()
            if prefetch:
                for dsc in gather_descs(li + 2, (k + 2) % _NSLOT):
                    dsc.start()
            if zdrain:
                zwrite_desc(li, k).wait()
            compute_chunk(k)
            zwrite_desc(li, k).start()

        for dsc in gather_descs(0, 0) + gather_descs(1, 1):
            dsc.start()
        for k in range(_NSLOT):
            step(k, k, zdrain=False, prefetch=True)

        def body(t, carry):
            for k in range(_NSLOT):
                step(_NSLOT * t + k, k, zdrain=True, prefetch=True)
            return carry

        lax.fori_loop(1, NLOC // _NSLOT - 1, body, 0)
        done = (NLOC // _NSLOT - 1) * _NSLOT
        for li in range(done, NLOC):
            k = li % _NSLOT
            step(li, k, zdrain=True, prefetch=(li + 2 < NLOC))
        for li in range(NLOC - _NSLOT, NLOC):
            zwrite_desc(li, li % _NSLOT).wait()

    return edge_head_kernel


_MBLK = 2000
_EBLK = 4000
_LSMBLK = 6400


def _dis_block(d0_ref, d1_ref):
    deg = d0_ref[:, 0:1] + d1_ref[:, 0:1] + 1.0
    return lax.rsqrt(deg)


def _mm1_body(x_ref, w_ref, d0_ref, d1_ref, o_ref):
    dis = _dis_block(d0_ref, d1_ref)
    xw = jnp.dot(x_ref[...], w_ref[...], preferred_element_type=jnp.float32)
    o_ref[...] = xw * dis


def _combine1_body(s0_ref, s1_ref, y_ref, d0_ref, d1_ref, b_ref, w_ref, o_ref):
    dis = _dis_block(d0_ref, d1_ref)
    h = jnp.maximum((s0_ref[...] + s1_ref[...] + y_ref[...]) * dis + b_ref[...], 0.0)
    o_ref[...] = jnp.dot(h, w_ref[...], preferred_element_type=jnp.float32) * dis


def _combine2_body(s0_ref, s1_ref, y_ref, d0_ref, d1_ref, b_ref, wl1_ref,
                   bl1_ref, oa_ref, ob_ref):
    dis = _dis_block(d0_ref, d1_ref)
    h = jnp.maximum((s0_ref[...] + s1_ref[...] + y_ref[...]) * dis + b_ref[...], 0.0)
    oa_ref[...] = (jnp.dot(h, wl1_ref[0:D_HID, :], preferred_element_type=jnp.float32)
                   + bl1_ref[...])
    ob_ref[...] = jnp.dot(h, wl1_ref[D_HID:2 * D_HID, :],
                          preferred_element_type=jnp.float32)


def _lsm_body(z_ref, o_ref):
    z0 = z_ref[0:1, :]
    z1 = z_ref[1:2, :]
    m = jnp.maximum(z0, z1)
    lse = m + jnp.log(jnp.exp(z0 - m) + jnp.exp(z1 - m))
    o_ref[...] = jnp.concatenate([z0 - lse, z1 - lse], axis=0)


def _node_spec(width):
    return pl.BlockSpec((_MBLK, width), lambda i: (i, 0))


def _full_spec(shape):
    return pl.BlockSpec(shape, lambda i: tuple(0 for _ in shape))


def kernel(x, edge_index, W1, b1, W2, b2, Wl1, bl1, Wl2, bl2):
    src = edge_index[0].astype(jnp.int32)
    dst = edge_index[1].astype(jnp.int32)

    npad = E_PAD - E
    pad_src = (jnp.arange(npad, dtype=jnp.int32) * 37) % N
    pad_dst = N + (jnp.arange(npad, dtype=jnp.int32) % PAD_BIN)
    srcp = jnp.concatenate([src, pad_src])
    dstp = jnp.concatenate([dst, pad_dst])
    dstp2d = dstp.reshape(NCHUNKS_PAD, CHUNK)

    zeros_deg = jnp.zeros((N, DEG_W), jnp.float32)
    ones_deg = jnp.ones((CHUNK, DEG_W), jnp.float32)
    zeros_hid = jnp.zeros((N, D_HID), jnp.float32)
    b1r = b1.reshape(1, D_HID)
    b2r = b2.reshape(1, D_HID)
    bl1r = bl1.reshape(1, D_HID)
    bl2r = bl2.reshape(1, 2)

    deg0, deg1 = _deg_sc()(dstp2d, ones_deg, zeros_deg)

    y1 = pl.pallas_call(
        _mm1_body,
        grid=(N // _MBLK,),
        in_specs=[
            _node_spec(D_IN),
            _full_spec((D_IN, D_HID)),
            _node_spec(DEG_W),
            _node_spec(DEG_W),
        ],
        out_specs=_node_spec(D_HID),
        out_shape=jax.ShapeDtypeStruct((N, D_HID), jnp.float32),
    )(x, W1, deg0, deg1)

    s1a, s1b = _msg_sc()(y1, srcp, dstp2d, zeros_hid)

    y2 = pl.pallas_call(
        _combine1_body,
        grid=(N // _MBLK,),
        in_specs=[
            _node_spec(D_HID),
            _node_spec(D_HID),
            _node_spec(D_HID),
            _node_spec(DEG_W),
            _node_spec(DEG_W),
            _full_spec((1, D_HID)),
            _full_spec((D_HID, D_HID)),
        ],
        out_specs=_node_spec(D_HID),
        out_shape=jax.ShapeDtypeStruct((N, D_HID), jnp.float32),
    )(s1a, s1b, y1, deg0, deg1, b1r, W2)

    s2a, s2b = _msg_sc()(y2, srcp, dstp2d, zeros_hid)

    ha, hb = pl.pallas_call(
        _combine2_body,
        grid=(N // _MBLK,),
        in_specs=[
            _node_spec(D_HID),
            _node_spec(D_HID),
            _node_spec(D_HID),
            _node_spec(DEG_W),
            _node_spec(DEG_W),
            _full_spec((1, D_HID)),
            _full_spec((D_IN, D_HID)),
            _full_spec((1, D_HID)),
        ],
        out_specs=(_node_spec(D_HID), _node_spec(D_HID)),
        out_shape=(
            jax.ShapeDtypeStruct((N, D_HID), jnp.float32),
            jax.ShapeDtypeStruct((N, D_HID), jnp.float32),
        ),
    )(s2a, s2b, y2, deg0, deg1, b2r, Wl1, bl1r)

    wl2b = jnp.broadcast_to(Wl2.T[:, :, None], (2, D_HID, 16)).astype(jnp.float32)
    bl2b = jnp.broadcast_to(bl2[:, None], (2, 16)).astype(jnp.float32)
    z = _edge_head_sc()(ha, hb, srcp, dstp, wl2b, bl2b)

    out2 = pl.pallas_call(
        _lsm_body,
        grid=(E // _LSMBLK,),
        in_specs=[pl.BlockSpec((2, _LSMBLK), lambda i: (0, i))],
        out_specs=pl.BlockSpec((2, _LSMBLK), lambda i: (0, i)),
        out_shape=jax.ShapeDtypeStruct((2, E), jnp.float32),
    )(z)

    return out2.T


# speedup vs baseline: 1.0436x; 1.0008x over previous
import functools

import jax
import jax.numpy as jnp
from jax import lax
from jax.experimental import pallas as pl
from jax.experimental.pallas import tpu as pltpu
from jax.experimental.pallas import tpu_sc as plsc

N = 10000
E = 320000
D_IN = 128
D_HID = 64

NC = 2
NS = 16
NW = NC * NS
CHUNK = 128
NLOC = 80
NCHUNKS_PAD = NW * NLOC
E_PAD = NCHUNKS_PAD * CHUNK
PAD_BIN = 16
N_ACC = N + PAD_BIN
ROWS_PER_SUB = 624
TAIL_ROWS = N - NS * ROWS_PER_SUB
DEG_W = 16

G_MSG = 4
NG_MSG = NLOC // G_MSG
G_PAIR = 2
NG_PAIR = NLOC // G_PAIR


def _worker_id():
    return lax.axis_index("s") * NC + lax.axis_index("c")


def _copy_share(src, dst, s):
    r0 = s * ROWS_PER_SUB
    pltpu.sync_copy(src.at[pl.ds(r0, ROWS_PER_SUB)],
                    dst.at[pl.ds(r0, ROWS_PER_SUB)])

    @pl.when(s == NS - 1)
    def _():
        t0 = NS * ROWS_PER_SUB
        pltpu.sync_copy(src.at[pl.ds(t0, TAIL_ROWS)],
                        dst.at[pl.ds(t0, TAIL_ROWS)])


def _writeback(acc, out0, out1, c, s):
    @pl.when(c == 0)
    def _():
        _copy_share(acc, out0, s)

    @pl.when(c == 1)
    def _():
        _copy_share(acc, out1, s)


@functools.lru_cache(maxsize=None)
def _sc_mesh():
    return plsc.VectorSubcoreMesh(
        core_axis_name="c", subcore_axis_name="s", num_cores=NC, num_subcores=NS
    )


@functools.lru_cache(maxsize=None)
def _deg_sc():
    @functools.partial(
        pl.kernel,
        out_type=(
            jax.ShapeDtypeStruct((N, DEG_W), jnp.float32),
            jax.ShapeDtypeStruct((N, DEG_W), jnp.float32),
        ),
        mesh=_sc_mesh(),
        scratch_types=[
            pltpu.VMEM((NLOC, CHUNK), jnp.int32),
            pltpu.VMEM((CHUNK, DEG_W), jnp.float32),
            pltpu.VMEM_SHARED((N_ACC, DEG_W), jnp.float32),
            pltpu.SemaphoreType.DMA,
        ],
        compiler_params=pltpu.CompilerParams(use_tc_tiling_on_sc=False),
    )
    def deg_kernel(dst2d_hbm, ones_hbm, zeros_hbm, out0, out1,
                   didx, ones_v, acc, sem):
        c = lax.axis_index("c")
        s = lax.axis_index("s")
        w = _worker_id()
        c0 = pl.multiple_of(w * NLOC, 8)
        pltpu.sync_copy(dst2d_hbm.at[pl.ds(c0, NLOC)], didx)
        pltpu.sync_copy(ones_hbm, ones_v)
        _copy_share(zeros_hbm, acc, s)
        plsc.subcore_barrier()

        def fire16(t, carry):
            for j in range(16):
                li = t * 16 + j
                pltpu.make_async_copy(ones_v, acc.at[didx.at[li]], sem).start(add=True)
            return carry

        def drain16(t, carry):
            for j in range(16):
                li = t * 16 + j
                pltpu.make_async_copy(ones_v, acc.at[didx.at[li]], sem).wait()
            return carry

        lax.fori_loop(0, NLOC // 16, fire16, 0)
        lax.fori_loop(0, NLOC // 16, drain16, 0)
        plsc.subcore_barrier()
        _writeback(acc, out0, out1, c, s)

    return deg_kernel


@functools.lru_cache(maxsize=None)
def _msg_sc():
    @functools.partial(
        pl.kernel,
        out_type=(
            jax.ShapeDtypeStruct((N, D_HID), jnp.float32),
            jax.ShapeDtypeStruct((N, D_HID), jnp.float32),
        ),
        mesh=_sc_mesh(),
        scratch_types=[
            pltpu.VMEM((NLOC * CHUNK,), jnp.int32),
            pltpu.VMEM((NLOC, CHUNK), jnp.int32),
            pltpu.VMEM((2 * G_MSG, CHUNK, D_HID), jnp.float32),
            pltpu.VMEM_SHARED((N_ACC, D_HID), jnp.float32),
            pltpu.SemaphoreType.DMA,
            pltpu.SemaphoreType.DMA,
            pltpu.SemaphoreType.DMA,
            pltpu.SemaphoreType.DMA,
        ],
        compiler_params=pltpu.CompilerParams(use_tc_tiling_on_sc=False),
    )
    def msg_kernel(y_hbm, src1d_hbm, dst2d_hbm, zeros_hbm, out0, out1,
                   sidx, didx, rows, acc, sga, sgb, ssa, ssb):
        c = lax.axis_index("c")
        s = lax.axis_index("s")
        w = _worker_id()
        c0 = pl.multiple_of(w * NLOC, 8)
        pltpu.sync_copy(src1d_hbm.at[pl.ds(c0 * CHUNK, NLOC * CHUNK)], sidx)
        pltpu.sync_copy(dst2d_hbm.at[pl.ds(c0, NLOC)], didx)
        _copy_share(zeros_hbm, acc, s)

        def gather_desc(g, bank, j, sem):
            li = g * G_MSG + j
            return pltpu.make_async_copy(
                y_hbm.at[sidx.at[pl.ds(li * CHUNK, CHUNK)]],
                rows.at[bank * G_MSG + j], sem)

        def scatter_desc(g, bank, j, sem):
            li = g * G_MSG + j
            return pltpu.make_async_copy(
                rows.at[bank * G_MSG + j], acc.at[didx.at[li]], sem)

        def fire_gathers(g, bank, sem):
            for j in range(G_MSG):
                gather_desc(g, bank, j, sem).start()

        def drain_gathers(g, bank, sem):
            for j in range(G_MSG):
                gather_desc(g, bank, j, sem).wait()

        def fire_scatters(g, bank, sem):
            for j in range(G_MSG):
                scatter_desc(g, bank, j, sem).start(add=True)

        def drain_scatters(g, bank, sem):
            for j in range(G_MSG):
                scatter_desc(g, bank, j, sem).wait()

        fire_gathers(0, 0, sga)
        fire_gathers(1, 1, sgb)
        plsc.subcore_barrier()

        def body(t, carry):
            g0 = 2 * t
            g1 = g0 + 1
            drain_gathers(g0, 0, sga)
            fire_scatters(g0, 0, ssa)
            drain_gathers(g1, 1, sgb)
            fire_scatters(g1, 1, ssb)
            drain_scatters(g0, 0, ssa)
            fire_gathers(g0 + 2, 0, sga)
            drain_scatters(g1, 1, ssb)
            fire_gathers(g1 + 2, 1, sgb)
            return carry

        lax.fori_loop(0, NG_MSG // 2 - 1, body, 0)
        g0 = NG_MSG - 2
        g1 = NG_MSG - 1
        drain_gathers(g0, 0, sga)
        fire_scatters(g0, 0, ssa)
        drain_gathers(g1, 1, sgb)
        fire_scatters(g1, 1, ssb)
        drain_scatters(g0, 0, ssa)
        drain_scatters(g1, 1, ssb)
        plsc.subcore_barrier()
        _writeback(acc, out0, out1, c, s)

    return msg_kernel


_NSLOT = 3
_GRP = CHUNK // 16
_TSTRIDE = 73


@functools.lru_cache(maxsize=None)
def _edge_head_sc():
    slot_scratch = []
    for _ in range(_NSLOT):
        slot_scratch += [
            pltpu.VMEM((CHUNK, D_HID), jnp.float32),
            pltpu.VMEM((CHUNK, D_HID), jnp.float32),
            pltpu.VMEM((CHUNK * _TSTRIDE,), jnp.float32),
            pltpu.VMEM((2, CHUNK), jnp.float32),
            pltpu.SemaphoreType.DMA,
            pltpu.SemaphoreType.DMA,
        ]

    @functools.partial(
        pl.kernel,
        out_type=jax.ShapeDtypeStruct((2, E_PAD), jnp.float32),
        mesh=_sc_mesh(),
        scratch_types=[
            pltpu.VMEM((NLOC * CHUNK,), jnp.int32),
            pltpu.VMEM((NLOC * CHUNK,), jnp.int32),
            pltpu.VMEM((2, D_HID, 16), jnp.float32),
            pltpu.VMEM((2, 16), jnp.float32),
        ] + slot_scratch,
        compiler_params=pltpu.CompilerParams(use_tc_tiling_on_sc=False,
                                             needs_layout_passes=False),
    )
    def edge_head_kernel(ha_hbm, hb_hbm, src1d_hbm, dst1d_hbm, wl2_hbm, bl2_hbm,
                         z_out, sidx, didx, wl2_v, bl2_v, *slot_refs):
        slots = [tuple(slot_refs[i * 6:(i + 1) * 6]) for i in range(_NSLOT)]
        w = _worker_id()
        c0 = pl.multiple_of(w * NLOC, 8)
        pltpu.sync_copy(src1d_hbm.at[pl.ds(c0 * CHUNK, NLOC * CHUNK)], sidx)
        pltpu.sync_copy(dst1d_hbm.at[pl.ds(c0 * CHUNK, NLOC * CHUNK)], didx)
        pltpu.sync_copy(wl2_hbm, wl2_v)
        pltpu.sync_copy(bl2_hbm, bl2_v)

        iota16 = lax.iota(jnp.int32, 16)
        col_idx = [iota16 * _TSTRIDE + (g * 16 * _TSTRIDE) for g in range(_GRP)]

        def gather_descs(li, k):
            ra, rb, _, _, sg, _ = slots[k]
            return (
                pltpu.make_async_copy(
                    ha_hbm.at[sidx.at[pl.ds(li * CHUNK, CHUNK)]], ra, sg),
                pltpu.make_async_copy(
                    hb_hbm.at[didx.at[pl.ds(li * CHUNK, CHUNK)]], rb, sg),
            )

        def zwrite_desc(li, k):
            _, _, _, zt, _, sw = slots[k]
            base = (c0 + li) * CHUNK
            return pltpu.make_async_copy(
                zt, z_out.at[:, pl.ds(base, CHUNK)], sw)

        def compute_chunk(k):
            ra, rb, tb, zt, _, _ = slots[k]

            def ebody(e8, carry):
                for u in range(8):
                    e = e8 * 8 + u
                    for q in range(D_HID // 16):
                        av = ra[e, pl.ds(q * 16, 16)]
                        bv = rb[e, pl.ds(q * 16, 16)]
                        tb[pl.ds(e * _TSTRIDE + q * 16, 16)] = jnp.maximum(
                            av + bv, 0.0)
                return carry

            lax.fori_loop(0, CHUNK // 8, ebody, 0)

            def dbody(d2, accs):
                out = list(accs)
                for dd in range(2):
                    d = d2 * 2 + dd
                    w0d = wl2_v[0, d]
                    w1d = wl2_v[1, d]
                    nxt = []
                    for g in range(_GRP):
                        r = plsc.load_gather(tb, [col_idx[g] + d])
                        nxt.append(out[2 * g] + r * w0d)
                        nxt.append(out[2 * g + 1] + r * w1d)
                    out = nxt
                return tuple(out)

            init = tuple(bl2_v[cc] for _ in range(_GRP) for cc in (0, 1))
            accs = lax.fori_loop(0, D_HID // 2, dbody, init)
            for g in range(_GRP):
                zt[0, pl.ds(g * 16, 16)] = accs[2 * g]
                zt[1, pl.ds(g * 16, 16)] = accs[2 * g + 1]

        def step(li, k, zdrain, prefetch):
            for dsc in gather_descs(li, k):
                dsc.wait()
            if prefetch:
                for dsc in gather_descs(li + 2, (k + 2) % _NSLOT):
                    dsc.start()
            if zdrain:
                zwrite_desc(li, k).wait()
            compute_chunk(k)
            zwrite_desc(li, k).start()

        for dsc in gather_descs(0, 0) + gather_descs(1, 1):
            dsc.start()
        for k in range(_NSLOT):
            step(k, k, zdrain=False, prefetch=True)

        def body(t, carry):
            for k in range(_NSLOT):
                step(_NSLOT * t + k, k, zdrain=True, prefetch=True)
            return carry

        lax.fori_loop(1, NLOC // _NSLOT - 1, body, 0)
        done = (NLOC // _NSLOT - 1) * _NSLOT
        for li in range(done, NLOC):
            k = li % _NSLOT
            step(li, k, zdrain=True, prefetch=(li + 2 < NLOC))
        for li in range(NLOC - _NSLOT, NLOC):
            zwrite_desc(li, li % _NSLOT).wait()

    return edge_head_kernel


_MBLK = 2000
_EBLK = 4000
_LSMBLK = 6400


def _dis_block(d0_ref, d1_ref):
    deg = d0_ref[:, 0:1] + d1_ref[:, 0:1] + 1.0
    return lax.rsqrt(deg)


def _mm1_body(x_ref, w_ref, d0_ref, d1_ref, o_ref):
    dis = _dis_block(d0_ref, d1_ref)
    xw = jnp.dot(x_ref[...], w_ref[...], preferred_element_type=jnp.float32)
    o_ref[...] = xw * dis


def _combine1_body(s0_ref, s1_ref, y_ref, d0_ref, d1_ref, b_ref, w_ref, o_ref):
    dis = _dis_block(d0_ref, d1_ref)
    h = jnp.maximum((s0_ref[...] + s1_ref[...] + y_ref[...]) * dis + b_ref[...], 0.0)
    o_ref[...] = jnp.dot(h, w_ref[...], preferred_element_type=jnp.float32) * dis


def _combine2_body(s0_ref, s1_ref, y_ref, d0_ref, d1_ref, b_ref, wl1_ref,
                   bl1_ref, oa_ref, ob_ref):
    dis = _dis_block(d0_ref, d1_ref)
    h = jnp.maximum((s0_ref[...] + s1_ref[...] + y_ref[...]) * dis + b_ref[...], 0.0)
    oa_ref[...] = (jnp.dot(h, wl1_ref[0:D_HID, :], preferred_element_type=jnp.float32)
                   + bl1_ref[...])
    ob_ref[...] = jnp.dot(h, wl1_ref[D_HID:2 * D_HID, :],
                          preferred_element_type=jnp.float32)


def _lsm_body(z_ref, o_ref):
    z0 = z_ref[0:1, :]
    z1 = z_ref[1:2, :]
    m = jnp.maximum(z0, z1)
    lse = m + jnp.log(jnp.exp(z0 - m) + jnp.exp(z1 - m))
    o_ref[...] = jnp.concatenate([z0 - lse, z1 - lse], axis=0)


def _node_spec(width):
    return pl.BlockSpec((_MBLK, width), lambda i: (i, 0))


def _full_spec(shape):
    return pl.BlockSpec(shape, lambda i: tuple(0 for _ in shape))


def kernel(x, edge_index, W1, b1, W2, b2, Wl1, bl1, Wl2, bl2):
    src = edge_index[0].astype(jnp.int32)
    dst = edge_index[1].astype(jnp.int32)

    npad = E_PAD - E
    pad_src = (jnp.arange(npad, dtype=jnp.int32) * 37) % N
    pad_dst = N + (jnp.arange(npad, dtype=jnp.int32) % PAD_BIN)
    srcp = jnp.concatenate([src, pad_src])
    dstp = jnp.concatenate([dst, pad_dst])
    dstp2d = dstp.reshape(NCHUNKS_PAD, CHUNK)

    zeros_deg = jnp.zeros((N, DEG_W), jnp.float32)
    ones_deg = jnp.ones((CHUNK, DEG_W), jnp.float32)
    zeros_hid = jnp.zeros((N, D_HID), jnp.float32)
    b1r = b1.reshape(1, D_HID)
    b2r = b2.reshape(1, D_HID)
    bl1r = bl1.reshape(1, D_HID)
    bl2r = bl2.reshape(1, 2)

    deg0, deg1 = _deg_sc()(dstp2d, ones_deg, zeros_deg)

    y1 = pl.pallas_call(
        _mm1_body,
        grid=(N // _MBLK,),
        in_specs=[
            _node_spec(D_IN),
            _full_spec((D_IN, D_HID)),
            _node_spec(DEG_W),
            _node_spec(DEG_W),
        ],
        out_specs=_node_spec(D_HID),
        out_shape=jax.ShapeDtypeStruct((N, D_HID), jnp.float32),
    )(x, W1, deg0, deg1)

    s1a, s1b = _msg_sc()(y1, srcp, dstp2d, zeros_hid)

    y2 = pl.pallas_call(
        _combine1_body,
        grid=(N // _MBLK,),
        in_specs=[
            _node_spec(D_HID),
            _node_spec(D_HID),
            _node_spec(D_HID),
            _node_spec(DEG_W),
            _node_spec(DEG_W),
            _full_spec((1, D_HID)),
            _full_spec((D_HID, D_HID)),
        ],
        out_specs=_node_spec(D_HID),
        out_shape=jax.ShapeDtypeStruct((N, D_HID), jnp.float32),
    )(s1a, s1b, y1, deg0, deg1, b1r, W2)

    s2a, s2b = _msg_sc()(y2, srcp, dstp2d, zeros_hid)

    ha, hb = pl.pallas_call(
        _combine2_body,
        grid=(N // _MBLK,),
        in_specs=[
            _node_spec(D_HID),
            _node_spec(D_HID),
            _node_spec(D_HID),
            _node_spec(DEG_W),
            _node_spec(DEG_W),
            _full_spec((1, D_HID)),
            _full_spec((D_IN, D_HID)),
            _full_spec((1, D_HID)),
        ],
        out_specs=(_node_spec(D_HID), _node_spec(D_HID)),
        out_shape=(
            jax.ShapeDtypeStruct((N, D_HID), jnp.float32),
            jax.ShapeDtypeStruct((N, D_HID), jnp.float32),
        ),
    )(s2a, s2b, y2, deg0, deg1, b2r, Wl1, bl1r)

    wl2b = jnp.broadcast_to(Wl2.T[:, :, None], (2, D_HID, 16)).astype(jnp.float32)
    bl2b = jnp.broadcast_to(bl2[:, None], (2, 16)).astype(jnp.float32)
    z = _edge_head_sc()(ha, hb, srcp, dstp, wl2b, bl2b)

    out2 = pl.pallas_call(
        _lsm_body,
        grid=(E // _LSMBLK,),
        in_specs=[pl.BlockSpec((2, _LSMBLK), lambda i: (0, i))],
        out_specs=pl.BlockSpec((2, _LSMBLK), lambda i: (0, i)),
        out_shape=jax.ShapeDtypeStruct((2, E), jnp.float32),
    )(z)

    return out2.T


# speedup vs baseline: 1.5482x; 1.4836x over previous
import functools

import jax
import jax.numpy as jnp
from jax import lax
from jax.experimental import pallas as pl
from jax.experimental.pallas import tpu as pltpu
from jax.experimental.pallas import tpu_sc as plsc

N = 10000
E = 320000
D_IN = 128
D_HID = 64

NC = 2
NS = 16
NW = NC * NS
CHUNK = 128
NLOC = 80
NCHUNKS_PAD = NW * NLOC
E_PAD = NCHUNKS_PAD * CHUNK
PAD_BIN = 16
N_ACC = N + PAD_BIN
ROWS_PER_SUB = 624
TAIL_ROWS = N - NS * ROWS_PER_SUB
DEG_W = 16

G_MSG = 4
NG_MSG = NLOC // G_MSG
G_PAIR = 2
NG_PAIR = NLOC // G_PAIR


def _worker_id():
    return lax.axis_index("s") * NC + lax.axis_index("c")


def _copy_share(src, dst, s):
    r0 = s * ROWS_PER_SUB
    pltpu.sync_copy(src.at[pl.ds(r0, ROWS_PER_SUB)],
                    dst.at[pl.ds(r0, ROWS_PER_SUB)])

    @pl.when(s == NS - 1)
    def _():
        t0 = NS * ROWS_PER_SUB
        pltpu.sync_copy(src.at[pl.ds(t0, TAIL_ROWS)],
                        dst.at[pl.ds(t0, TAIL_ROWS)])


def _writeback(acc, out0, out1, c, s):
    @pl.when(c == 0)
    def _():
        _copy_share(acc, out0, s)

    @pl.when(c == 1)
    def _():
        _copy_share(acc, out1, s)


@functools.lru_cache(maxsize=None)
def _sc_mesh():
    return plsc.VectorSubcoreMesh(
        core_axis_name="c", subcore_axis_name="s", num_cores=NC, num_subcores=NS
    )


@functools.lru_cache(maxsize=None)
def _deg_sc():
    @functools.partial(
        pl.kernel,
        out_type=(
            jax.ShapeDtypeStruct((N, DEG_W), jnp.float32),
            jax.ShapeDtypeStruct((N, DEG_W), jnp.float32),
        ),
        mesh=_sc_mesh(),
        scratch_types=[
            pltpu.VMEM((NLOC, CHUNK), jnp.int32),
            pltpu.VMEM((CHUNK, DEG_W), jnp.float32),
            pltpu.VMEM_SHARED((N_ACC, DEG_W), jnp.float32),
            pltpu.SemaphoreType.DMA,
        ],
        compiler_params=pltpu.CompilerParams(use_tc_tiling_on_sc=False),
    )
    def deg_kernel(dst2d_hbm, ones_hbm, zeros_hbm, out0, out1,
                   didx, ones_v, acc, sem):
        c = lax.axis_index("c")
        s = lax.axis_index("s")
        w = _worker_id()
        c0 = pl.multiple_of(w * NLOC, 8)
        pltpu.sync_copy(dst2d_hbm.at[pl.ds(c0, NLOC)], didx)
        pltpu.sync_copy(ones_hbm, ones_v)
        _copy_share(zeros_hbm, acc, s)
        plsc.subcore_barrier()

        def fire16(t, carry):
            for j in range(16):
                li = t * 16 + j
                pltpu.make_async_copy(ones_v, acc.at[didx.at[li]], sem).start(add=True)
            return carry

        def drain16(t, carry):
            for j in range(16):
                li = t * 16 + j
                pltpu.make_async_copy(ones_v, acc.at[didx.at[li]], sem).wait()
            return carry

        lax.fori_loop(0, NLOC // 16, fire16, 0)
        lax.fori_loop(0, NLOC // 16, drain16, 0)
        plsc.subcore_barrier()
        _writeback(acc, out0, out1, c, s)

    return deg_kernel


@functools.lru_cache(maxsize=None)
def _msg_sc():
    @functools.partial(
        pl.kernel,
        out_type=(
            jax.ShapeDtypeStruct((N, D_HID), jnp.float32),
            jax.ShapeDtypeStruct((N, D_HID), jnp.float32),
        ),
        mesh=_sc_mesh(),
        scratch_types=[
            pltpu.VMEM((NLOC * CHUNK,), jnp.int32),
            pltpu.VMEM((NLOC, CHUNK), jnp.int32),
            pltpu.VMEM((2 * G_MSG, CHUNK, D_HID), jnp.float32),
            pltpu.VMEM_SHARED((N_ACC, D_HID), jnp.float32),
            pltpu.SemaphoreType.DMA,
            pltpu.SemaphoreType.DMA,
            pltpu.SemaphoreType.DMA,
            pltpu.SemaphoreType.DMA,
        ],
        compiler_params=pltpu.CompilerParams(use_tc_tiling_on_sc=False),
    )
    def msg_kernel(y_hbm, src1d_hbm, dst2d_hbm, zeros_hbm, out0, out1,
                   sidx, didx, rows, acc, sga, sgb, ssa, ssb):
        c = lax.axis_index("c")
        s = lax.axis_index("s")
        w = _worker_id()
        c0 = pl.multiple_of(w * NLOC, 8)
        pltpu.sync_copy(src1d_hbm.at[pl.ds(c0 * CHUNK, NLOC * CHUNK)], sidx)
        pltpu.sync_copy(dst2d_hbm.at[pl.ds(c0, NLOC)], didx)
        _copy_share(zeros_hbm, acc, s)

        def gather_desc(g, bank, j, sem):
            li = g * G_MSG + j
            return pltpu.make_async_copy(
                y_hbm.at[sidx.at[pl.ds(li * CHUNK, CHUNK)]],
                rows.at[bank * G_MSG + j], sem)

        def scatter_desc(g, bank, j, sem):
            li = g * G_MSG + j
            return pltpu.make_async_copy(
                rows.at[bank * G_MSG + j], acc.at[didx.at[li]], sem)

        def fire_gathers(g, bank, sem):
            for j in range(G_MSG):
                gather_desc(g, bank, j, sem).start()

        def drain_gathers(g, bank, sem):
            for j in range(G_MSG):
                gather_desc(g, bank, j, sem).wait()

        def fire_scatters(g, bank, sem):
            for j in range(G_MSG):
                scatter_desc(g, bank, j, sem).start(add=True)

        def drain_scatters(g, bank, sem):
            for j in range(G_MSG):
                scatter_desc(g, bank, j, sem).wait()

        fire_gathers(0, 0, sga)
        fire_gathers(1, 1, sgb)
        plsc.subcore_barrier()

        def body(t, carry):
            g0 = 2 * t
            g1 = g0 + 1
            drain_gathers(g0, 0, sga)
            fire_scatters(g0, 0, ssa)
            drain_gathers(g1, 1, sgb)
            fire_scatters(g1, 1, ssb)
            drain_scatters(g0, 0, ssa)
            fire_gathers(g0 + 2, 0, sga)
            drain_scatters(g1, 1, ssb)
            fire_gathers(g1 + 2, 1, sgb)
            return carry

        lax.fori_loop(0, NG_MSG // 2 - 1, body, 0)
        g0 = NG_MSG - 2
        g1 = NG_MSG - 1
        drain_gathers(g0, 0, sga)
        fire_scatters(g0, 0, ssa)
        drain_gathers(g1, 1, sgb)
        fire_scatters(g1, 1, ssb)
        drain_scatters(g0, 0, ssa)
        drain_scatters(g1, 1, ssb)
        plsc.subcore_barrier()
        _writeback(acc, out0, out1, c, s)

    return msg_kernel


_NSLOT = 3
_GRP = CHUNK // 16
_TSTRIDE = 73


@functools.lru_cache(maxsize=None)
def _edge_head_sc():
    slot_scratch = []
    for _ in range(_NSLOT):
        slot_scratch += [
            pltpu.VMEM((CHUNK, D_HID), jnp.float32),
            pltpu.VMEM((CHUNK, D_HID), jnp.float32),
            pltpu.VMEM((2, CHUNK), jnp.float32),
            pltpu.SemaphoreType.DMA,
            pltpu.SemaphoreType.DMA,
        ]

    @functools.partial(
        pl.kernel,
        out_type=jax.ShapeDtypeStruct((2, E_PAD), jnp.float32),
        mesh=_sc_mesh(),
        scratch_types=[
            pltpu.VMEM((NLOC * CHUNK,), jnp.int32),
            pltpu.VMEM((NLOC * CHUNK,), jnp.int32),
            pltpu.VMEM((2, D_HID, 16), jnp.float32),
            pltpu.VMEM((2, 16), jnp.float32),
        ] + slot_scratch,
        compiler_params=pltpu.CompilerParams(use_tc_tiling_on_sc=False,
                                             needs_layout_passes=False),
    )
    def edge_head_kernel(ha_hbm, hb_hbm, src1d_hbm, dst1d_hbm, wl2_hbm, bl2_hbm,
                         z_out, sidx, didx, wl2_v, bl2_v, *slot_refs):
        slots = [tuple(slot_refs[i * 5:(i + 1) * 5]) for i in range(_NSLOT)]
        w = _worker_id()
        c0 = pl.multiple_of(w * NLOC, 8)
        pltpu.sync_copy(src1d_hbm.at[pl.ds(c0 * CHUNK, NLOC * CHUNK)], sidx)
        pltpu.sync_copy(dst1d_hbm.at[pl.ds(c0 * CHUNK, NLOC * CHUNK)], didx)
        pltpu.sync_copy(wl2_hbm, wl2_v)
        pltpu.sync_copy(bl2_hbm, bl2_v)

        iota16 = lax.iota(jnp.int32, 16)
        row_idx = [iota16 + g * 16 for g in range(_GRP)]

        def gather_descs(li, k):
            ra, rb, _, sg, _ = slots[k]
            return (
                pltpu.make_async_copy(
                    ha_hbm.at[sidx.at[pl.ds(li * CHUNK, CHUNK)]], ra, sg),
                pltpu.make_async_copy(
                    hb_hbm.at[didx.at[pl.ds(li * CHUNK, CHUNK)]], rb, sg),
            )

        def zwrite_desc(li, k):
            _, _, zt, _, sw = slots[k]
            base = (c0 + li) * CHUNK
            return pltpu.make_async_copy(
                zt, z_out.at[:, pl.ds(base, CHUNK)], sw)

        def compute_chunk(k):
            ra, rb, zt, _, _ = slots[k]

            def kbody(kk, accs):
                w0k = wl2_v[0, kk]
                w1k = wl2_v[1, kk]
                dsel = jnp.bitwise_and(iota16 + kk, D_HID - 1)
                out = []
                for g in range(_GRP):
                    av = plsc.load_gather(ra, [row_idx[g], dsel])
                    bv = plsc.load_gather(rb, [row_idx[g], dsel])
                    r = jnp.maximum(av + bv, 0.0)
                    out.append(accs[2 * g] + r * w0k)
                    out.append(accs[2 * g + 1] + r * w1k)
                return tuple(out)

            init = tuple(bl2_v[cc] for _ in range(_GRP) for cc in (0, 1))
            accs = lax.fori_loop(0, D_HID, kbody, init)
            for g in range(_GRP):
                zt[0, pl.ds(g * 16, 16)] = accs[2 * g]
                zt[1, pl.ds(g * 16, 16)] = accs[2 * g + 1]

        def step(li, k, zdrain, prefetch):
            for dsc in gather_descs(li, k):
                dsc.wait()
            if prefetch:
                for dsc in gather_descs(li + 2, (k + 2) % _NSLOT):
                    dsc.start()
            if zdrain:
                zwrite_desc(li, k).wait()
            compute_chunk(k)
            zwrite_desc(li, k).start()

        for dsc in gather_descs(0, 0) + gather_descs(1, 1):
            dsc.start()
        for k in range(_NSLOT):
            step(k, k, zdrain=False, prefetch=True)

        def body(t, carry):
            for k in range(_NSLOT):
                step(_NSLOT * t + k, k, zdrain=True, prefetch=True)
            return carry

        lax.fori_loop(1, NLOC // _NSLOT - 1, body, 0)
        done = (NLOC // _NSLOT - 1) * _NSLOT
        for li in range(done, NLOC):
            k = li % _NSLOT
            step(li, k, zdrain=True, prefetch=(li + 2 < NLOC))
        for li in range(NLOC - _NSLOT, NLOC):
            zwrite_desc(li, li % _NSLOT).wait()

    return edge_head_kernel


_MBLK = 2000
_EBLK = 4000
_LSMBLK = 6400


def _dis_block(d0_ref, d1_ref):
    deg = d0_ref[:, 0:1] + d1_ref[:, 0:1] + 1.0
    return lax.rsqrt(deg)


def _mm1_body(x_ref, w_ref, d0_ref, d1_ref, o_ref):
    dis = _dis_block(d0_ref, d1_ref)
    xw = jnp.dot(x_ref[...], w_ref[...], preferred_element_type=jnp.float32)
    o_ref[...] = xw * dis


def _combine1_body(s0_ref, s1_ref, y_ref, d0_ref, d1_ref, b_ref, w_ref, o_ref):
    dis = _dis_block(d0_ref, d1_ref)
    h = jnp.maximum((s0_ref[...] + s1_ref[...] + y_ref[...]) * dis + b_ref[...], 0.0)
    o_ref[...] = jnp.dot(h, w_ref[...], preferred_element_type=jnp.float32) * dis


def _combine2_body(s0_ref, s1_ref, y_ref, d0_ref, d1_ref, b_ref, wl1_ref,
                   bl1_ref, oa_ref, ob_ref):
    dis = _dis_block(d0_ref, d1_ref)
    h = jnp.maximum((s0_ref[...] + s1_ref[...] + y_ref[...]) * dis + b_ref[...], 0.0)
    oa_ref[...] = (jnp.dot(h, wl1_ref[0:D_HID, :], preferred_element_type=jnp.float32)
                   + bl1_ref[...])
    ob_ref[...] = jnp.dot(h, wl1_ref[D_HID:2 * D_HID, :],
                          preferred_element_type=jnp.float32)


def _lsm_body(z_ref, o_ref):
    z0 = z_ref[0:1, :]
    z1 = z_ref[1:2, :]
    m = jnp.maximum(z0, z1)
    lse = m + jnp.log(jnp.exp(z0 - m) + jnp.exp(z1 - m))
    o_ref[...] = jnp.concatenate([z0 - lse, z1 - lse], axis=0)


def _node_spec(width):
    return pl.BlockSpec((_MBLK, width), lambda i: (i, 0))


def _full_spec(shape):
    return pl.BlockSpec(shape, lambda i: tuple(0 for _ in shape))


def kernel(x, edge_index, W1, b1, W2, b2, Wl1, bl1, Wl2, bl2):
    src = edge_index[0].astype(jnp.int32)
    dst = edge_index[1].astype(jnp.int32)

    npad = E_PAD - E
    pad_src = (jnp.arange(npad, dtype=jnp.int32) * 37) % N
    pad_dst = N + (jnp.arange(npad, dtype=jnp.int32) % PAD_BIN)
    srcp = jnp.concatenate([src, pad_src])
    dstp = jnp.concatenate([dst, pad_dst])
    dstp2d = dstp.reshape(NCHUNKS_PAD, CHUNK)

    zeros_deg = jnp.zeros((N, DEG_W), jnp.float32)
    ones_deg = jnp.ones((CHUNK, DEG_W), jnp.float32)
    zeros_hid = jnp.zeros((N, D_HID), jnp.float32)
    b1r = b1.reshape(1, D_HID)
    b2r = b2.reshape(1, D_HID)
    bl1r = bl1.reshape(1, D_HID)
    bl2r = bl2.reshape(1, 2)

    deg0, deg1 = _deg_sc()(dstp2d, ones_deg, zeros_deg)

    y1 = pl.pallas_call(
        _mm1_body,
        grid=(N // _MBLK,),
        in_specs=[
            _node_spec(D_IN),
            _full_spec((D_IN, D_HID)),
            _node_spec(DEG_W),
            _node_spec(DEG_W),
        ],
        out_specs=_node_spec(D_HID),
        out_shape=jax.ShapeDtypeStruct((N, D_HID), jnp.float32),
    )(x, W1, deg0, deg1)

    s1a, s1b = _msg_sc()(y1, srcp, dstp2d, zeros_hid)

    y2 = pl.pallas_call(
        _combine1_body,
        grid=(N // _MBLK,),
        in_specs=[
            _node_spec(D_HID),
            _node_spec(D_HID),
            _node_spec(D_HID),
            _node_spec(DEG_W),
            _node_spec(DEG_W),
            _full_spec((1, D_HID)),
            _full_spec((D_HID, D_HID)),
        ],
        out_specs=_node_spec(D_HID),
        out_shape=jax.ShapeDtypeStruct((N, D_HID), jnp.float32),
    )(s1a, s1b, y1, deg0, deg1, b1r, W2)

    s2a, s2b = _msg_sc()(y2, srcp, dstp2d, zeros_hid)

    ha, hb = pl.pallas_call(
        _combine2_body,
        grid=(N // _MBLK,),
        in_specs=[
            _node_spec(D_HID),
            _node_spec(D_HID),
            _node_spec(D_HID),
            _node_spec(DEG_W),
            _node_spec(DEG_W),
            _full_spec((1, D_HID)),
            _full_spec((D_IN, D_HID)),
            _full_spec((1, D_HID)),
        ],
        out_specs=(_node_spec(D_HID), _node_spec(D_HID)),
        out_shape=(
            jax.ShapeDtypeStruct((N, D_HID), jnp.float32),
            jax.ShapeDtypeStruct((N, D_HID), jnp.float32),
        ),
    )(s2a, s2b, y2, deg0, deg1, b2r, Wl1, bl1r)

    rot = (jnp.arange(D_HID)[:, None] + jnp.arange(16)[None, :]) % D_HID
    wl2b = jnp.transpose(Wl2[rot, :], (2, 0, 1)).astype(jnp.float32)
    bl2b = jnp.broadcast_to(bl2[:, None], (2, 16)).astype(jnp.float32)
    z = _edge_head_sc()(ha, hb, srcp, dstp, wl2b, bl2b)

    out2 = pl.pallas_call(
        _lsm_body,
        grid=(E // _LSMBLK,),
        in_specs=[pl.BlockSpec((2, _LSMBLK), lambda i: (0, i))],
        out_specs=pl.BlockSpec((2, _LSMBLK), lambda i: (0, i)),
        out_shape=jax.ShapeDtypeStruct((2, E), jnp.float32),
    )(z)

    return out2.T


# speedup vs baseline: 1.6115x; 1.0409x over previous
import functools

import jax
import jax.numpy as jnp
from jax import lax
from jax.experimental import pallas as pl
from jax.experimental.pallas import tpu as pltpu
from jax.experimental.pallas import tpu_sc as plsc

N = 10000
E = 320000
D_IN = 128
D_HID = 64

NC = 2
NS = 16
NW = NC * NS
CHUNK = 128
NLOC = 80
NCHUNKS_PAD = NW * NLOC
E_PAD = NCHUNKS_PAD * CHUNK
PAD_BIN = 16
N_ACC = N + PAD_BIN
ROWS_PER_SUB = 624
TAIL_ROWS = N - NS * ROWS_PER_SUB
DEG_W = 16

G_MSG = 4
NG_MSG = NLOC // G_MSG
G_PAIR = 2
NG_PAIR = NLOC // G_PAIR


def _worker_id():
    return lax.axis_index("s") * NC + lax.axis_index("c")


def _copy_share(src, dst, s):
    r0 = s * ROWS_PER_SUB
    pltpu.sync_copy(src.at[pl.ds(r0, ROWS_PER_SUB)],
                    dst.at[pl.ds(r0, ROWS_PER_SUB)])

    @pl.when(s == NS - 1)
    def _():
        t0 = NS * ROWS_PER_SUB
        pltpu.sync_copy(src.at[pl.ds(t0, TAIL_ROWS)],
                        dst.at[pl.ds(t0, TAIL_ROWS)])


def _writeback(acc, out0, out1, c, s):
    @pl.when(c == 0)
    def _():
        _copy_share(acc, out0, s)

    @pl.when(c == 1)
    def _():
        _copy_share(acc, out1, s)


@functools.lru_cache(maxsize=None)
def _sc_mesh():
    return plsc.VectorSubcoreMesh(
        core_axis_name="c", subcore_axis_name="s", num_cores=NC, num_subcores=NS
    )


@functools.lru_cache(maxsize=None)
def _deg_sc():
    @functools.partial(
        pl.kernel,
        out_type=(
            jax.ShapeDtypeStruct((N, DEG_W), jnp.float32),
            jax.ShapeDtypeStruct((N, DEG_W), jnp.float32),
        ),
        mesh=_sc_mesh(),
        scratch_types=[
            pltpu.VMEM((NLOC, CHUNK), jnp.int32),
            pltpu.VMEM((CHUNK, DEG_W), jnp.float32),
            pltpu.VMEM_SHARED((N_ACC, DEG_W), jnp.float32),
            pltpu.SemaphoreType.DMA,
        ],
        compiler_params=pltpu.CompilerParams(use_tc_tiling_on_sc=False),
    )
    def deg_kernel(dst2d_hbm, ones_hbm, zeros_hbm, out0, out1,
                   didx, ones_v, acc, sem):
        c = lax.axis_index("c")
        s = lax.axis_index("s")
        w = _worker_id()
        c0 = pl.multiple_of(w * NLOC, 8)
        pltpu.sync_copy(dst2d_hbm.at[pl.ds(c0, NLOC)], didx)
        pltpu.sync_copy(ones_hbm, ones_v)
        _copy_share(zeros_hbm, acc, s)
        plsc.subcore_barrier()

        def fire16(t, carry):
            for j in range(16):
                li = t * 16 + j
                pltpu.make_async_copy(ones_v, acc.at[didx.at[li]], sem).start(add=True)
            return carry

        def drain16(t, carry):
            for j in range(16):
                li = t * 16 + j
                pltpu.make_async_copy(ones_v, acc.at[didx.at[li]], sem).wait()
            return carry

        lax.fori_loop(0, NLOC // 16, fire16, 0)
        lax.fori_loop(0, NLOC // 16, drain16, 0)
        plsc.subcore_barrier()
        _writeback(acc, out0, out1, c, s)

    return deg_kernel


@functools.lru_cache(maxsize=None)
def _msg_sc():
    @functools.partial(
        pl.kernel,
        out_type=(
            jax.ShapeDtypeStruct((N, D_HID), jnp.float32),
            jax.ShapeDtypeStruct((N, D_HID), jnp.float32),
        ),
        mesh=_sc_mesh(),
        scratch_types=[
            pltpu.VMEM((NLOC * CHUNK,), jnp.int32),
            pltpu.VMEM((NLOC, CHUNK), jnp.int32),
            pltpu.VMEM((2 * G_MSG, CHUNK, D_HID), jnp.float32),
            pltpu.VMEM_SHARED((N_ACC, D_HID), jnp.float32),
            pltpu.SemaphoreType.DMA,
            pltpu.SemaphoreType.DMA,
            pltpu.SemaphoreType.DMA,
            pltpu.SemaphoreType.DMA,
        ],
        compiler_params=pltpu.CompilerParams(use_tc_tiling_on_sc=False),
    )
    def msg_kernel(y_hbm, src1d_hbm, dst2d_hbm, zeros_hbm, out0, out1,
                   sidx, didx, rows, acc, sga, sgb, ssa, ssb):
        c = lax.axis_index("c")
        s = lax.axis_index("s")
        w = _worker_id()
        c0 = pl.multiple_of(w * NLOC, 8)
        pltpu.sync_copy(src1d_hbm.at[pl.ds(c0 * CHUNK, NLOC * CHUNK)], sidx)
        pltpu.sync_copy(dst2d_hbm.at[pl.ds(c0, NLOC)], didx)
        _copy_share(zeros_hbm, acc, s)

        def gather_desc(g, bank, j, sem):
            li = g * G_MSG + j
            return pltpu.make_async_copy(
                y_hbm.at[sidx.at[pl.ds(li * CHUNK, CHUNK)]],
                rows.at[bank * G_MSG + j], sem)

        def scatter_desc(g, bank, j, sem):
            li = g * G_MSG + j
            return pltpu.make_async_copy(
                rows.at[bank * G_MSG + j], acc.at[didx.at[li]], sem)

        def fire_gathers(g, bank, sem):
            for j in range(G_MSG):
                gather_desc(g, bank, j, sem).start()

        def drain_gathers(g, bank, sem):
            for j in range(G_MSG):
                gather_desc(g, bank, j, sem).wait()

        def fire_scatters(g, bank, sem):
            for j in range(G_MSG):
                scatter_desc(g, bank, j, sem).start(add=True)

        def drain_scatters(g, bank, sem):
            for j in range(G_MSG):
                scatter_desc(g, bank, j, sem).wait()

        fire_gathers(0, 0, sga)
        fire_gathers(1, 1, sgb)
        plsc.subcore_barrier()

        def body(t, carry):
            g0 = 2 * t
            g1 = g0 + 1
            drain_gathers(g0, 0, sga)
            fire_scatters(g0, 0, ssa)
            drain_gathers(g1, 1, sgb)
            fire_scatters(g1, 1, ssb)
            drain_scatters(g0, 0, ssa)
            fire_gathers(g0 + 2, 0, sga)
            drain_scatters(g1, 1, ssb)
            fire_gathers(g1 + 2, 1, sgb)
            return carry

        lax.fori_loop(0, NG_MSG // 2 - 1, body, 0)
        g0 = NG_MSG - 2
        g1 = NG_MSG - 1
        drain_gathers(g0, 0, sga)
        fire_scatters(g0, 0, ssa)
        drain_gathers(g1, 1, sgb)
        fire_scatters(g1, 1, ssb)
        drain_scatters(g0, 0, ssa)
        drain_scatters(g1, 1, ssb)
        plsc.subcore_barrier()
        _writeback(acc, out0, out1, c, s)

    return msg_kernel


_NSLOT = 4
_GRP = CHUNK // 16
_TSTRIDE = 73


@functools.lru_cache(maxsize=None)
def _edge_head_sc():
    slot_scratch = []
    for _ in range(_NSLOT):
        slot_scratch += [
            pltpu.VMEM((CHUNK, D_HID), jnp.float32),
            pltpu.VMEM((CHUNK, D_HID), jnp.float32),
            pltpu.VMEM((2, CHUNK), jnp.float32),
            pltpu.SemaphoreType.DMA,
            pltpu.SemaphoreType.DMA,
        ]

    @functools.partial(
        pl.kernel,
        out_type=jax.ShapeDtypeStruct((2, E_PAD), jnp.float32),
        mesh=_sc_mesh(),
        scratch_types=[
            pltpu.VMEM((NLOC * CHUNK,), jnp.int32),
            pltpu.VMEM((NLOC * CHUNK,), jnp.int32),
            pltpu.VMEM((2, D_HID, 16), jnp.float32),
            pltpu.VMEM((2, 16), jnp.float32),
        ] + slot_scratch,
        compiler_params=pltpu.CompilerParams(use_tc_tiling_on_sc=False,
                                             needs_layout_passes=False),
    )
    def edge_head_kernel(ha_hbm, hb_hbm, src1d_hbm, dst1d_hbm, wl2_hbm, bl2_hbm,
                         z_out, sidx, didx, wl2_v, bl2_v, *slot_refs):
        slots = [tuple(slot_refs[i * 5:(i + 1) * 5]) for i in range(_NSLOT)]
        w = _worker_id()
        c0 = pl.multiple_of(w * NLOC, 8)
        pltpu.sync_copy(src1d_hbm.at[pl.ds(c0 * CHUNK, NLOC * CHUNK)], sidx)
        pltpu.sync_copy(dst1d_hbm.at[pl.ds(c0 * CHUNK, NLOC * CHUNK)], didx)
        pltpu.sync_copy(wl2_hbm, wl2_v)
        pltpu.sync_copy(bl2_hbm, bl2_v)

        iota16 = lax.iota(jnp.int32, 16)
        row_idx = [iota16 + g * 16 for g in range(_GRP)]

        def gather_descs(li, k):
            ra, rb, _, sg, _ = slots[k]
            return (
                pltpu.make_async_copy(
                    ha_hbm.at[sidx.at[pl.ds(li * CHUNK, CHUNK)]], ra, sg),
                pltpu.make_async_copy(
                    hb_hbm.at[didx.at[pl.ds(li * CHUNK, CHUNK)]], rb, sg),
            )

        def zwrite_desc(li, k):
            _, _, zt, _, sw = slots[k]
            base = (c0 + li) * CHUNK
            return pltpu.make_async_copy(
                zt, z_out.at[:, pl.ds(base, CHUNK)], sw)

        def compute_chunk(k):
            ra, rb, zt, _, _ = slots[k]

            def kbody(kk, accs):
                w0k = wl2_v[0, kk]
                w1k = wl2_v[1, kk]
                dsel = jnp.bitwise_and(iota16 + kk, D_HID - 1)
                out = []
                for g in range(_GRP):
                    av = plsc.load_gather(ra, [row_idx[g], dsel])
                    bv = plsc.load_gather(rb, [row_idx[g], dsel])
                    r = jnp.maximum(av + bv, 0.0)
                    out.append(accs[2 * g] + r * w0k)
                    out.append(accs[2 * g + 1] + r * w1k)
                return tuple(out)

            init = tuple(bl2_v[cc] for _ in range(_GRP) for cc in (0, 1))
            accs = lax.fori_loop(0, D_HID, kbody, init)
            for g in range(_GRP):
                zt[0, pl.ds(g * 16, 16)] = accs[2 * g]
                zt[1, pl.ds(g * 16, 16)] = accs[2 * g + 1]

        def step(li, k, zdrain, prefetch):
            for dsc in gather_descs(li, k):
                dsc.wait()
            if prefetch:
                for dsc in gather_descs(li + 3, (k + 3) % _NSLOT):
                    dsc.start()
            if zdrain:
                zwrite_desc(li, k).wait()
            compute_chunk(k)
            zwrite_desc(li, k).start()

        for dsc in gather_descs(0, 0) + gather_descs(1, 1) + gather_descs(2, 2):
            dsc.start()
        for k in range(_NSLOT):
            step(k, k, zdrain=False, prefetch=True)

        def body(t, carry):
            for k in range(_NSLOT):
                step(_NSLOT * t + k, k, zdrain=True, prefetch=True)
            return carry

        lax.fori_loop(1, NLOC // _NSLOT - 1, body, 0)
        done = (NLOC // _NSLOT - 1) * _NSLOT
        for li in range(done, NLOC):
            k = li % _NSLOT
            step(li, k, zdrain=True, prefetch=(li + 3 < NLOC))
        for li in range(NLOC - _NSLOT, NLOC):
            zwrite_desc(li, li % _NSLOT).wait()

    return edge_head_kernel


_MBLK = 2000
_EBLK = 4000
_LSMBLK = 8192


def _dis_block(d0_ref, d1_ref):
    deg = d0_ref[:, 0:1] + d1_ref[:, 0:1] + 1.0
    return lax.rsqrt(deg)


def _mm1_body(x_ref, w_ref, d0_ref, d1_ref, o_ref):
    dis = _dis_block(d0_ref, d1_ref)
    xw = jnp.dot(x_ref[...], w_ref[...], preferred_element_type=jnp.float32)
    o_ref[...] = xw * dis


def _combine1_body(s0_ref, s1_ref, y_ref, d0_ref, d1_ref, b_ref, w_ref, o_ref):
    dis = _dis_block(d0_ref, d1_ref)
    h = jnp.maximum((s0_ref[...] + s1_ref[...] + y_ref[...]) * dis + b_ref[...], 0.0)
    o_ref[...] = jnp.dot(h, w_ref[...], preferred_element_type=jnp.float32) * dis


def _combine2_body(s0_ref, s1_ref, y_ref, d0_ref, d1_ref, b_ref, wl1_ref,
                   bl1_ref, oa_ref, ob_ref):
    dis = _dis_block(d0_ref, d1_ref)
    h = jnp.maximum((s0_ref[...] + s1_ref[...] + y_ref[...]) * dis + b_ref[...], 0.0)
    oa_ref[...] = (jnp.dot(h, wl1_ref[0:D_HID, :], preferred_element_type=jnp.float32)
                   + bl1_ref[...])
    ob_ref[...] = jnp.dot(h, wl1_ref[D_HID:2 * D_HID, :],
                          preferred_element_type=jnp.float32)


def _lsm_body(z_ref, o_ref):
    z0 = z_ref[0:8, :]
    z1 = z_ref[8:16, :]
    m = jnp.maximum(z0, z1)
    lse = m + jnp.log(jnp.exp(z0 - m) + jnp.exp(z1 - m))
    o_ref[...] = jnp.concatenate([z0 - lse, z1 - lse], axis=0)


def _node_spec(width):
    return pl.BlockSpec((_MBLK, width), lambda i: (i, 0))


def _full_spec(shape):
    return pl.BlockSpec(shape, lambda i: tuple(0 for _ in shape))


def kernel(x, edge_index, W1, b1, W2, b2, Wl1, bl1, Wl2, bl2):
    src = edge_index[0].astype(jnp.int32)
    dst = edge_index[1].astype(jnp.int32)

    npad = E_PAD - E
    pad_src = jnp.arange(npad, dtype=jnp.int32) & 8191
    pad_dst = N + (jnp.arange(npad, dtype=jnp.int32) & (PAD_BIN - 1))
    srcp = jnp.concatenate([src, pad_src])
    dstp = jnp.concatenate([dst, pad_dst])
    dstp2d = dstp.reshape(NCHUNKS_PAD, CHUNK)

    zeros_deg = jnp.zeros((N, DEG_W), jnp.float32)
    ones_deg = jnp.ones((CHUNK, DEG_W), jnp.float32)
    zeros_hid = jnp.zeros((N, D_HID), jnp.float32)
    b1r = b1.reshape(1, D_HID)
    b2r = b2.reshape(1, D_HID)
    bl1r = bl1.reshape(1, D_HID)
    bl2r = bl2.reshape(1, 2)

    deg0, deg1 = _deg_sc()(dstp2d, ones_deg, zeros_deg)

    y1 = pl.pallas_call(
        _mm1_body,
        grid=(N // _MBLK,),
        in_specs=[
            _node_spec(D_IN),
            _full_spec((D_IN, D_HID)),
            _node_spec(DEG_W),
            _node_spec(DEG_W),
        ],
        out_specs=_node_spec(D_HID),
        out_shape=jax.ShapeDtypeStruct((N, D_HID), jnp.float32),
    )(x, W1, deg0, deg1)

    s1a, s1b = _msg_sc()(y1, srcp, dstp2d, zeros_hid)

    y2 = pl.pallas_call(
        _combine1_body,
        grid=(N // _MBLK,),
        in_specs=[
            _node_spec(D_HID),
            _node_spec(D_HID),
            _node_spec(D_HID),
            _node_spec(DEG_W),
            _node_spec(DEG_W),
            _full_spec((1, D_HID)),
            _full_spec((D_HID, D_HID)),
        ],
        out_specs=_node_spec(D_HID),
        out_shape=jax.ShapeDtypeStruct((N, D_HID), jnp.float32),
    )(s1a, s1b, y1, deg0, deg1, b1r, W2)

    s2a, s2b = _msg_sc()(y2, srcp, dstp2d, zeros_hid)

    ha, hb = pl.pallas_call(
        _combine2_body,
        grid=(N // _MBLK,),
        in_specs=[
            _node_spec(D_HID),
            _node_spec(D_HID),
            _node_spec(D_HID),
            _node_spec(DEG_W),
            _node_spec(DEG_W),
            _full_spec((1, D_HID)),
            _full_spec((D_IN, D_HID)),
            _full_spec((1, D_HID)),
        ],
        out_specs=(_node_spec(D_HID), _node_spec(D_HID)),
        out_shape=(
            jax.ShapeDtypeStruct((N, D_HID), jnp.float32),
            jax.ShapeDtypeStruct((N, D_HID), jnp.float32),
        ),
    )(s2a, s2b, y2, deg0, deg1, b2r, Wl1, bl1r)

    rot = (jnp.arange(D_HID)[:, None] + jnp.arange(16)[None, :]) % D_HID
    wl2b = jnp.transpose(Wl2[rot, :], (2, 0, 1)).astype(jnp.float32)
    bl2b = jnp.broadcast_to(bl2[:, None], (2, 16)).astype(jnp.float32)
    z = _edge_head_sc()(ha, hb, srcp, dstp, wl2b, bl2b)

    z16 = z.reshape(16, E_PAD // 8)
    out16 = pl.pallas_call(
        _lsm_body,
        grid=(E_PAD // 8 // _LSMBLK,),
        in_specs=[pl.BlockSpec((16, _LSMBLK), lambda i: (0, i))],
        out_specs=pl.BlockSpec((16, _LSMBLK), lambda i: (0, i)),
        out_shape=jax.ShapeDtypeStruct((16, E_PAD // 8), jnp.float32),
    )(z16)

    return out16.reshape(2, E_PAD)[:, :E].T


# speedup vs baseline: 1.7625x; 1.0937x over previous
import functools

import jax
import jax.numpy as jnp
from jax import lax
from jax.experimental import pallas as pl
from jax.experimental.pallas import tpu as pltpu
from jax.experimental.pallas import tpu_sc as plsc

N = 10000
E = 320000
D_IN = 128
D_HID = 64

NC = 2
NS = 16
NW = NC * NS
CHUNK = 128
NLOC = 80
NCHUNKS_PAD = NW * NLOC
E_PAD = NCHUNKS_PAD * CHUNK
PAD_BIN = 16
N_ACC = N + PAD_BIN
ROWS_PER_SUB = 624
TAIL_ROWS = N - NS * ROWS_PER_SUB
DEG_W = 16

G_MSG = 4
NG_MSG = NLOC // G_MSG
G_PAIR = 2
NG_PAIR = NLOC // G_PAIR


def _worker_id():
    return lax.axis_index("s") * NC + lax.axis_index("c")


def _copy_share(src, dst, s):
    r0 = s * ROWS_PER_SUB
    pltpu.sync_copy(src.at[pl.ds(r0, ROWS_PER_SUB)],
                    dst.at[pl.ds(r0, ROWS_PER_SUB)])

    @pl.when(s == NS - 1)
    def _():
        t0 = NS * ROWS_PER_SUB
        pltpu.sync_copy(src.at[pl.ds(t0, TAIL_ROWS)],
                        dst.at[pl.ds(t0, TAIL_ROWS)])


def _writeback(acc, out0, out1, c, s):
    @pl.when(c == 0)
    def _():
        _copy_share(acc, out0, s)

    @pl.when(c == 1)
    def _():
        _copy_share(acc, out1, s)


@functools.lru_cache(maxsize=None)
def _sc_mesh():
    return plsc.VectorSubcoreMesh(
        core_axis_name="c", subcore_axis_name="s", num_cores=NC, num_subcores=NS
    )


@functools.lru_cache(maxsize=None)
def _deg_sc():
    @functools.partial(
        pl.kernel,
        out_type=(
            jax.ShapeDtypeStruct((N, DEG_W), jnp.float32),
            jax.ShapeDtypeStruct((N, DEG_W), jnp.float32),
        ),
        mesh=_sc_mesh(),
        scratch_types=[
            pltpu.VMEM((NLOC, CHUNK), jnp.int32),
            pltpu.VMEM((CHUNK, DEG_W), jnp.float32),
            pltpu.VMEM_SHARED((N_ACC, DEG_W), jnp.float32),
            pltpu.SemaphoreType.DMA,
        ],
        compiler_params=pltpu.CompilerParams(use_tc_tiling_on_sc=False),
    )
    def deg_kernel(dst2d_hbm, ones_hbm, zeros_hbm, out0, out1,
                   didx, ones_v, acc, sem):
        c = lax.axis_index("c")
        s = lax.axis_index("s")
        w = _worker_id()
        c0 = pl.multiple_of(w * NLOC, 8)
        pltpu.sync_copy(dst2d_hbm.at[pl.ds(c0, NLOC)], didx)
        pltpu.sync_copy(ones_hbm, ones_v)
        _copy_share(zeros_hbm, acc, s)
        plsc.subcore_barrier()

        def fire16(t, carry):
            for j in range(16):
                li = t * 16 + j
                pltpu.make_async_copy(ones_v, acc.at[didx.at[li]], sem).start(add=True)
            return carry

        def drain16(t, carry):
            for j in range(16):
                li = t * 16 + j
                pltpu.make_async_copy(ones_v, acc.at[didx.at[li]], sem).wait()
            return carry

        lax.fori_loop(0, NLOC // 16, fire16, 0)
        lax.fori_loop(0, NLOC // 16, drain16, 0)
        plsc.subcore_barrier()
        _writeback(acc, out0, out1, c, s)

    return deg_kernel


@functools.lru_cache(maxsize=None)
def _msg_sc():
    slot_scratch = []
    for _ in range(4):
        slot_scratch += [
            pltpu.VMEM((CHUNK, D_HID), jnp.float32),
            pltpu.SemaphoreType.DMA,
            pltpu.SemaphoreType.DMA,
        ]

    @functools.partial(
        pl.kernel,
        out_type=(
            jax.ShapeDtypeStruct((N, D_HID), jnp.float32),
            jax.ShapeDtypeStruct((N, D_HID), jnp.float32),
        ),
        mesh=_sc_mesh(),
        scratch_types=[
            pltpu.VMEM((NLOC * CHUNK,), jnp.int32),
            pltpu.VMEM((NLOC, CHUNK), jnp.int32),
            pltpu.VMEM_SHARED((N_ACC, D_HID), jnp.float32),
        ] + slot_scratch,
        compiler_params=pltpu.CompilerParams(use_tc_tiling_on_sc=False),
    )
    def msg_kernel(y_hbm, src1d_hbm, dst2d_hbm, zeros_hbm, out0, out1,
                   sidx, didx, acc, *slot_refs):
        slots = [tuple(slot_refs[i * 3:(i + 1) * 3]) for i in range(4)]
        c = lax.axis_index("c")
        s = lax.axis_index("s")
        w = _worker_id()
        c0 = pl.multiple_of(w * NLOC, 8)
        pltpu.sync_copy(src1d_hbm.at[pl.ds(c0 * CHUNK, NLOC * CHUNK)], sidx)
        pltpu.sync_copy(dst2d_hbm.at[pl.ds(c0, NLOC)], didx)
        _copy_share(zeros_hbm, acc, s)

        def gather_desc(li, k):
            rows, sg, _ = slots[k]
            return pltpu.make_async_copy(
                y_hbm.at[sidx.at[pl.ds(li * CHUNK, CHUNK)]], rows, sg)

        def scatter_desc(li, k):
            rows, _, ss = slots[k]
            return pltpu.make_async_copy(rows, acc.at[didx.at[li]], ss)

        for k in range(3):
            gather_desc(k, k).start()
        plsc.subcore_barrier()

        def step(li, k, drain_prev, prefetch):
            gather_desc(li, k).wait()
            scatter_desc(li, k).start(add=True)
            if drain_prev:
                scatter_desc(li - 1, (k + 3) % 4).wait()
            if prefetch:
                gather_desc(li + 3, (k + 3) % 4).start()

        step(0, 0, drain_prev=False, prefetch=True)
        for li in range(1, 4):
            step(li, li, drain_prev=True, prefetch=True)

        def body(t, carry):
            for k in range(4):
                step(4 * t + k, k, drain_prev=True, prefetch=True)
            return carry

        lax.fori_loop(1, NLOC // 4 - 1, body, 0)
        for li in range(NLOC - 4, NLOC):
            step(li, li % 4, drain_prev=True, prefetch=(li + 3 < NLOC))
        scatter_desc(NLOC - 1, (NLOC - 1) % 4).wait()

        plsc.subcore_barrier()
        _writeback(acc, out0, out1, c, s)

    return msg_kernel


_NSLOT = 4
_GRP = CHUNK // 16
_TSTRIDE = 73


@functools.lru_cache(maxsize=None)
def _edge_head_sc():
    slot_scratch = []
    for _ in range(_NSLOT):
        slot_scratch += [
            pltpu.VMEM((CHUNK, D_HID), jnp.float32),
            pltpu.VMEM((CHUNK, D_HID), jnp.float32),
            pltpu.VMEM((2, CHUNK), jnp.float32),
            pltpu.SemaphoreType.DMA,
            pltpu.SemaphoreType.DMA,
        ]

    @functools.partial(
        pl.kernel,
        out_type=jax.ShapeDtypeStruct((2, E_PAD), jnp.float32),
        mesh=_sc_mesh(),
        scratch_types=[
            pltpu.VMEM((NLOC * CHUNK,), jnp.int32),
            pltpu.VMEM((NLOC * CHUNK,), jnp.int32),
            pltpu.VMEM((2, D_HID, 16), jnp.float32),
            pltpu.VMEM((2, 16), jnp.float32),
        ] + slot_scratch,
        compiler_params=pltpu.CompilerParams(use_tc_tiling_on_sc=False,
                                             needs_layout_passes=False),
    )
    def edge_head_kernel(ha_hbm, hb_hbm, src1d_hbm, dst1d_hbm, wl2_hbm, bl2_hbm,
                         z_out, sidx, didx, wl2_v, bl2_v, *slot_refs):
        slots = [tuple(slot_refs[i * 5:(i + 1) * 5]) for i in range(_NSLOT)]
        w = _worker_id()
        c0 = pl.multiple_of(w * NLOC, 8)
        pltpu.sync_copy(src1d_hbm.at[pl.ds(c0 * CHUNK, NLOC * CHUNK)], sidx)
        pltpu.sync_copy(dst1d_hbm.at[pl.ds(c0 * CHUNK, NLOC * CHUNK)], didx)
        pltpu.sync_copy(wl2_hbm, wl2_v)
        pltpu.sync_copy(bl2_hbm, bl2_v)

        iota16 = lax.iota(jnp.int32, 16)
        row_idx = [iota16 + g * 16 for g in range(_GRP)]

        def gather_descs(li, k):
            ra, rb, _, sg, _ = slots[k]
            return (
                pltpu.make_async_copy(
                    ha_hbm.at[sidx.at[pl.ds(li * CHUNK, CHUNK)]], ra, sg),
                pltpu.make_async_copy(
                    hb_hbm.at[didx.at[pl.ds(li * CHUNK, CHUNK)]], rb, sg),
            )

        def zwrite_desc(li, k):
            _, _, zt, _, sw = slots[k]
            base = (c0 + li) * CHUNK
            return pltpu.make_async_copy(
                zt, z_out.at[:, pl.ds(base, CHUNK)], sw)

        def compute_chunk(k):
            ra, rb, zt, _, _ = slots[k]

            def kbody(kk, accs):
                w0k = wl2_v[0, kk]
                w1k = wl2_v[1, kk]
                dsel = jnp.bitwise_and(iota16 + kk, D_HID - 1)
                out = []
                for g in range(_GRP):
                    av = plsc.load_gather(ra, [row_idx[g], dsel])
                    bv = plsc.load_gather(rb, [row_idx[g], dsel])
                    r = jnp.maximum(av + bv, 0.0)
                    out.append(accs[2 * g] + r * w0k)
                    out.append(accs[2 * g + 1] + r * w1k)
                return tuple(out)

            init = tuple(bl2_v[cc] for _ in range(_GRP) for cc in (0, 1))
            accs = lax.fori_loop(0, D_HID, kbody, init)
            for g in range(_GRP):
                zt[0, pl.ds(g * 16, 16)] = accs[2 * g]
                zt[1, pl.ds(g * 16, 16)] = accs[2 * g + 1]

        def step(li, k, zdrain, prefetch):
            for dsc in gather_descs(li, k):
                dsc.wait()
            if prefetch:
                for dsc in gather_descs(li + 3, (k + 3) % _NSLOT):
                    dsc.start()
            if zdrain:
                zwrite_desc(li, k).wait()
            compute_chunk(k)
            zwrite_desc(li, k).start()

        for dsc in gather_descs(0, 0) + gather_descs(1, 1) + gather_descs(2, 2):
            dsc.start()
        for k in range(_NSLOT):
            step(k, k, zdrain=False, prefetch=True)

        def body(t, carry):
            for k in range(_NSLOT):
                step(_NSLOT * t + k, k, zdrain=True, prefetch=True)
            return carry

        lax.fori_loop(1, NLOC // _NSLOT - 1, body, 0)
        done = (NLOC // _NSLOT - 1) * _NSLOT
        for li in range(done, NLOC):
            k = li % _NSLOT
            step(li, k, zdrain=True, prefetch=(li + 3 < NLOC))
        for li in range(NLOC - _NSLOT, NLOC):
            zwrite_desc(li, li % _NSLOT).wait()

    return edge_head_kernel


_MBLK = 2000
_EBLK = 4000
_LSMBLK = 8192


def _dis_block(d0_ref, d1_ref):
    deg = d0_ref[:, 0:1] + d1_ref[:, 0:1] + 1.0
    return lax.rsqrt(deg)


def _mm1_body(x_ref, w_ref, d0_ref, d1_ref, o_ref):
    dis = _dis_block(d0_ref, d1_ref)
    xw = jnp.dot(x_ref[...], w_ref[...], preferred_element_type=jnp.float32)
    o_ref[...] = xw * dis


def _combine1_body(s0_ref, s1_ref, y_ref, d0_ref, d1_ref, b_ref, w_ref, o_ref):
    dis = _dis_block(d0_ref, d1_ref)
    h = jnp.maximum((s0_ref[...] + s1_ref[...] + y_ref[...]) * dis + b_ref[...], 0.0)
    o_ref[...] = jnp.dot(h, w_ref[...], preferred_element_type=jnp.float32) * dis


def _combine2_body(s0_ref, s1_ref, y_ref, d0_ref, d1_ref, b_ref, wl1_ref,
                   bl1_ref, oa_ref, ob_ref):
    dis = _dis_block(d0_ref, d1_ref)
    h = jnp.maximum((s0_ref[...] + s1_ref[...] + y_ref[...]) * dis + b_ref[...], 0.0)
    oa_ref[...] = (jnp.dot(h, wl1_ref[0:D_HID, :], preferred_element_type=jnp.float32)
                   + bl1_ref[...])
    ob_ref[...] = jnp.dot(h, wl1_ref[D_HID:2 * D_HID, :],
                          preferred_element_type=jnp.float32)


def _lsm_body(z_ref, o_ref):
    z0 = z_ref[0:8, :]
    z1 = z_ref[8:16, :]
    m = jnp.maximum(z0, z1)
    lse = m + jnp.log(jnp.exp(z0 - m) + jnp.exp(z1 - m))
    o_ref[...] = jnp.concatenate([z0 - lse, z1 - lse], axis=0)


def _node_spec(width):
    return pl.BlockSpec((_MBLK, width), lambda i: (i, 0))


def _full_spec(shape):
    return pl.BlockSpec(shape, lambda i: tuple(0 for _ in shape))


def kernel(x, edge_index, W1, b1, W2, b2, Wl1, bl1, Wl2, bl2):
    src = edge_index[0].astype(jnp.int32)
    dst = edge_index[1].astype(jnp.int32)

    npad = E_PAD - E
    pad_src = jnp.arange(npad, dtype=jnp.int32) & 8191
    pad_dst = N + (jnp.arange(npad, dtype=jnp.int32) & (PAD_BIN - 1))
    srcp = jnp.concatenate([src, pad_src])
    dstp = jnp.concatenate([dst, pad_dst])
    dstp2d = dstp.reshape(NCHUNKS_PAD, CHUNK)

    zeros_deg = jnp.zeros((N, DEG_W), jnp.float32)
    ones_deg = jnp.ones((CHUNK, DEG_W), jnp.float32)
    zeros_hid = jnp.zeros((N, D_HID), jnp.float32)
    b1r = b1.reshape(1, D_HID)
    b2r = b2.reshape(1, D_HID)
    bl1r = bl1.reshape(1, D_HID)
    bl2r = bl2.reshape(1, 2)

    deg0, deg1 = _deg_sc()(dstp2d, ones_deg, zeros_deg)

    y1 = pl.pallas_call(
        _mm1_body,
        grid=(N // _MBLK,),
        in_specs=[
            _node_spec(D_IN),
            _full_spec((D_IN, D_HID)),
            _node_spec(DEG_W),
            _node_spec(DEG_W),
        ],
        out_specs=_node_spec(D_HID),
        out_shape=jax.ShapeDtypeStruct((N, D_HID), jnp.float32),
    )(x, W1, deg0, deg1)

    s1a, s1b = _msg_sc()(y1, srcp, dstp2d, zeros_hid)

    y2 = pl.pallas_call(
        _combine1_body,
        grid=(N // _MBLK,),
        in_specs=[
            _node_spec(D_HID),
            _node_spec(D_HID),
            _node_spec(D_HID),
            _node_spec(DEG_W),
            _node_spec(DEG_W),
            _full_spec((1, D_HID)),
            _full_spec((D_HID, D_HID)),
        ],
        out_specs=_node_spec(D_HID),
        out_shape=jax.ShapeDtypeStruct((N, D_HID), jnp.float32),
    )(s1a, s1b, y1, deg0, deg1, b1r, W2)

    s2a, s2b = _msg_sc()(y2, srcp, dstp2d, zeros_hid)

    ha, hb = pl.pallas_call(
        _combine2_body,
        grid=(N // _MBLK,),
        in_specs=[
            _node_spec(D_HID),
            _node_spec(D_HID),
            _node_spec(D_HID),
            _node_spec(DEG_W),
            _node_spec(DEG_W),
            _full_spec((1, D_HID)),
            _full_spec((D_IN, D_HID)),
            _full_spec((1, D_HID)),
        ],
        out_specs=(_node_spec(D_HID), _node_spec(D_HID)),
        out_shape=(
            jax.ShapeDtypeStruct((N, D_HID), jnp.float32),
            jax.ShapeDtypeStruct((N, D_HID), jnp.float32),
        ),
    )(s2a, s2b, y2, deg0, deg1, b2r, Wl1, bl1r)

    rot = (jnp.arange(D_HID)[:, None] + jnp.arange(16)[None, :]) % D_HID
    wl2b = jnp.transpose(Wl2[rot, :], (2, 0, 1)).astype(jnp.float32)
    bl2b = jnp.broadcast_to(bl2[:, None], (2, 16)).astype(jnp.float32)
    z = _edge_head_sc()(ha, hb, srcp, dstp, wl2b, bl2b)

    z16 = z.reshape(16, E_PAD // 8)
    out16 = pl.pallas_call(
        _lsm_body,
        grid=(E_PAD // 8 // _LSMBLK,),
        in_specs=[pl.BlockSpec((16, _LSMBLK), lambda i: (0, i))],
        out_specs=pl.BlockSpec((16, _LSMBLK), lambda i: (0, i)),
        out_shape=jax.ShapeDtypeStruct((16, E_PAD // 8), jnp.float32),
    )(z16)

    return out16.reshape(2, E_PAD)[:, :E].T
